# baseline (speedup 1.0000x reference)
import functools

import jax
import jax.numpy as jnp
from jax import lax
from jax.experimental import pallas as pl
from jax.experimental.pallas import tpu as pltpu
from jax.experimental.pallas import tpu_sc as plsc

N_NODES = 10000
N_CELLS = 10000
NODE_TYPE_SIZE = 9
LATENT = 128
MP_STEPS = 15

N_PAD = 10240
H = 3 * N_CELLS
H_PAD = 30720
E_PAD = 2 * H_PAD
NW = 32
NSLOT = 8

GPW = E_PAD // NW
GCH = GPW // 120
HALF_N = N_PAD // 2
ACC_ROWS = 5248
TRASH = ACC_ROWS - 1
SPW = E_PAD // 16
SCH = SPW // 120
SB = 240
SNB = SPW // SB

BN = 1024
BE = 1024


def _dot(a, w):
    return lax.dot_general(a, w, (((1,), (0,)), ((), ())),
                           preferred_element_type=jnp.float32)


def _layernorm(h, g, b):
    m = jnp.mean(h, axis=-1, keepdims=True)
    v = jnp.mean((h - m) ** 2, axis=-1, keepdims=True)
    return (h - m) * lax.rsqrt(v + 1e-5) * g + b


def _enc_body(x_ref, w1, b1, w2, b2, w3, b3, g, bl, o_ref):
    h = jnp.maximum(_dot(x_ref[...], w1[...]) + b1[...], 0.0)
    h = jnp.maximum(_dot(h, w2[...]) + b2[...], 0.0)
    h = _dot(h, w3[...]) + b3[...]
    o_ref[...] = _layernorm(h, g[...], bl[...])


def _dec_body(x_ref, w1, b1, w2, b2, w3, b3, o_ref):
    h = jnp.maximum(_dot(x_ref[...], w1[...]) + b1[...], 0.0)
    h = jnp.maximum(_dot(h, w2[...]) + b2[...], 0.0)
    o_ref[...] = _dot(h, w3[...]) + b3[...]


def _edge_body(ua, ub, e, mk, w1a, w1b, w1c, b1, w2, b2, w3, b3, g, bl,
               ne_ref, y_ref):
    a = ua[0]
    b = ub[0]
    e0 = e[0]
    h = (_dot(a, w1a[...]) + _dot(b, w1b[...]) + _dot(e0, w1c[...]) + b1[...])
    h = jnp.maximum(h, 0.0)
    h = jnp.maximum(_dot(h, w2[...]) + b2[...], 0.0)
    h = _dot(h, w3[...]) + b3[...]
    ne = _layernorm(h, g[...], bl[...]) + e0
    ne_ref[0] = ne
    y_ref[0] = ne * mk[...]


def _node_body(nd, ag, w1a, w1b, b1, w2, b2, w3, b3, g, bl, o_ref):
    h = jnp.maximum(_dot(nd[...], w1a[...]) + _dot(ag[...], w1b[...])
                    + b1[...], 0.0)
    h = jnp.maximum(_dot(h, w2[...]) + b2[...], 0.0)
    h = _dot(h, w3[...]) + b3[...]
    o_ref[...] = _layernorm(h, g[...], bl[...]) + nd[...]


def _rowspec(blk):
    return pl.BlockSpec((blk, LATENT), lambda i: (i, 0))


_WSPEC = pl.BlockSpec((LATENT, LATENT), lambda i: (0, 0))
_VSPEC = pl.BlockSpec((1, LATENT), lambda i: (0, 0))
_WSPEC2 = pl.BlockSpec((LATENT, LATENT), lambda h, i: (0, 0))
_VSPEC2 = pl.BlockSpec((1, LATENT), lambda h, i: (0, 0))


def _enc(x, w1, b1, w2, b2, w3, b3, g, bl, blk):
    rows = x.shape[0]
    return pl.pallas_call(
        _enc_body,
        grid=(rows // blk,),
        in_specs=[_rowspec(blk)] + [_WSPEC, _VSPEC] * 3 + [_VSPEC, _VSPEC],
        out_specs=_rowspec(blk),
        out_shape=jax.ShapeDtypeStruct((rows, LATENT), jnp.float32),
    )(x, w1, b1, w2, b2, w3, b3, g, bl)


def _dec(x, w1, b1, w2, b2, w3, b3, blk):
    rows = x.shape[0]
    return pl.pallas_call(
        _dec_body,
        grid=(rows // blk,),
        in_specs=[_rowspec(blk)] + [_WSPEC, _VSPEC] * 3,
        out_specs=_rowspec(blk),
        out_shape=jax.ShapeDtypeStruct((rows, LATENT), jnp.float32),
    )(x, w1, b1, w2, b2, w3, b3)


def _edge_step(u, edge3, mk, w1a, w1b, w1c, b1, w2, b2, w3, b3, g, bl):
    half = pl.BlockSpec((1, BE, LATENT), lambda h, i: (h, i, 0))
    swap = pl.BlockSpec((1, BE, LATENT), lambda h, i: (1 - h, i, 0))
    mspec = pl.BlockSpec((BE, 1), lambda h, i: (i, 0))
    return pl.pallas_call(
        _edge_body,
        grid=(2, H_PAD // BE),
        in_specs=[half, swap, half, mspec,
                  _WSPEC2, _WSPEC2, _WSPEC2, _VSPEC2,
                  _WSPEC2, _VSPEC2, _WSPEC2, _VSPEC2, _VSPEC2, _VSPEC2],
        out_specs=[half, half],
        out_shape=[jax.ShapeDtypeStruct((2, H_PAD, LATENT), jnp.float32),
                   jax.ShapeDtypeStruct((2, H_PAD, LATENT), jnp.float32)],
    )(u, u, edge3, mk, w1a, w1b, w1c, b1, w2, b2, w3, b3, g, bl)


def _node_step(node, agg, w1a, w1b, b1, w2, b2, w3, b3, g, bl):
    return pl.pallas_call(
        _node_body,
        grid=(N_PAD // BN,),
        in_specs=[_rowspec(BN), _rowspec(BN),
                  _WSPEC, _WSPEC, _VSPEC,
                  _WSPEC, _VSPEC, _WSPEC, _VSPEC, _VSPEC, _VSPEC],
        out_specs=_rowspec(BN),
        out_shape=jax.ShapeDtypeStruct((N_PAD, LATENT), jnp.float32),
    )(node, agg, w1a, w1b, b1, w2, b2, w3, b3, g, bl)


@functools.cache
def _build_sc_gather():
    mesh = plsc.VectorSubcoreMesh(core_axis_name="c", subcore_axis_name="s")

    @functools.partial(
        pl.kernel,
        mesh=mesh,
        out_type=jax.ShapeDtypeStruct((E_PAD, LATENT), jnp.float32),
        scratch_types=[pltpu.VMEM((GCH, 120), jnp.int32),
                       pltpu.VMEM((480, LATENT), jnp.float32),
                       pltpu.VMEM((480, LATENT), jnp.float32),
                       pltpu.SemaphoreType.DMA, pltpu.SemaphoreType.DMA,
                       pltpu.SemaphoreType.DMA, pltpu.SemaphoreType.DMA],
    )
    def gather_kernel(node_hbm, idx_hbm, out_hbm, idx_v, big0, big1,
                      gs0, gs1, ws0, ws1):
        bigs = (big0, big1)
        gsem = (gs0, gs1)
        wsem = (ws0, ws1)
        wid = lax.axis_index("s") * 2 + lax.axis_index("c")
        pltpu.sync_copy(idx_hbm.at[wid], idx_v)
        base = wid * GPW

        def _issue_batch(b, k):
            for j in range(4):
                pltpu.async_copy(node_hbm.at[idx_v.at[b * 4 + j]],
                                 bigs[k].at[pl.ds(j * 120, 120)], gsem[k])

        def _out_at(b):
            return out_hbm.at[pl.ds(base + b * 480, 480)]

        _issue_batch(0, 0)
        _issue_batch(1, 1)
        for b in range(4):
            k = b % 2
            pltpu.make_async_copy(_out_at(b), bigs[k], gsem[k]).wait()
            pltpu.async_copy(bigs[k], _out_at(b), wsem[k])
            n = b + 2
            if n < 4:
                pltpu.make_async_copy(bigs[k], _out_at(b), wsem[k]).wait()
                _issue_batch(n, k)
        for k in range(2):
            pltpu.make_async_copy(bigs[k], _out_at(2 + k), wsem[k]).wait()

    return gather_kernel


def _sc_gather(node, gidx):
    return _build_sc_gather()(node, gidx)


@functools.cache
def _build_sc_scatter():
    mesh = plsc.VectorSubcoreMesh(core_axis_name="c", subcore_axis_name="s")

    @functools.partial(
        pl.kernel,
        mesh=mesh,
        out_type=jax.ShapeDtypeStruct((2, HALF_N, LATENT), jnp.float32),
        scratch_types=[pltpu.VMEM((SCH, 120), jnp.int32),
                       pltpu.VMEM_SHARED((ACC_ROWS, LATENT), jnp.float32),
                       pltpu.VMEM((SB, LATENT), jnp.float32),
                       pltpu.VMEM((SB, LATENT), jnp.float32),
                       pltpu.SemaphoreType.DMA, pltpu.SemaphoreType.DMA,
                       pltpu.SemaphoreType.DMA, pltpu.SemaphoreType.DMA],
    )
    def scatter_kernel(y_hbm, idx_hbm, out_hbm, idx_v, shared, b0, b1,
                       ls0, ls1, ss0, ss1):
        bufs = (b0, b1)
        lsem = (ls0, ls1)
        ssem = (ss0, ss1)
        cc = lax.axis_index("c")
        s = lax.axis_index("s")

        def zrow(i, carry):
            for j in range(LATENT // 16):
                b0[i, pl.ds(j * 16, 16)] = jnp.zeros((16,), jnp.float32)
            return carry

        lax.fori_loop(0, SB, zrow, 0)
        z0 = s * (ACC_ROWS // 16)
        zcp = [pltpu.async_copy(b0, shared.at[pl.ds(z0 + off, SB)], lsem[i])
               for i, off in enumerate((0, ACC_ROWS // 16 - SB))]
        for d in zcp:
            d.wait()
        plsc.subcore_barrier()

        pltpu.sync_copy(idx_hbm.at[cc, s], idx_v)
        base = s * SPW

        def _y_at(b):
            return y_hbm.at[pl.ds(base + b * SB, SB)]

        def _adds_done(k):
            pltpu.make_async_copy(bufs[k], shared.at[pl.ds(0, SB)],
                                  ssem[k]).wait()

        for k in range(2):
            pltpu.async_copy(_y_at(k), bufs[k], lsem[k])

        def cycle(t, carry):
            for k in range(2):
                b = t * 2 + k
                pltpu.make_async_copy(_y_at(b), bufs[k], lsem[k]).wait()
                pltpu.async_copy(bufs[k].at[pl.ds(0, 120)],
                                 shared.at[idx_v.at[2 * b]], ssem[k],
                                 add=True)
                pltpu.async_copy(bufs[k].at[pl.ds(120, 120)],
                                 shared.at[idx_v.at[2 * b + 1]], ssem[k],
                                 add=True)
            for k in range(2):
                b = t * 2 + k
                n = b + 2

                @pl.when(n < SNB)
                def _():
                    _adds_done(k)
                    pltpu.async_copy(_y_at(n), bufs[k], lsem[k])
            return carry

        lax.fori_loop(0, SNB // 2, cycle, 0)
        for k in range(2):
            _adds_done(k)
        plsc.subcore_barrier()

        o0 = s * (HALF_N // 16)
        ocp = []
        for i, off in enumerate((0, HALF_N // 16 - SB)):
            pltpu.async_copy(shared.at[pl.ds(o0 + off, SB)],
                             bufs[i], lsem[i]).wait()
            ocp.append(pltpu.async_copy(
                bufs[i], out_hbm.at[cc, pl.ds(o0 + off, SB)], ssem[i]))
        for d in ocp:
            d.wait()

    return scatter_kernel


def _sc_scatter(y, sidx):
    return _build_sc_scatter()(y, sidx)


def _tri_edges(cells):
    cells = cells.astype(jnp.int32)
    e = jnp.concatenate([cells[:, 0:2], cells[:, 1:3],
                         jnp.stack([cells[:, 2], cells[:, 0]], axis=1)],
                        axis=0)
    lo = jnp.minimum(e[:, 0], e[:, 1])
    hi = jnp.maximum(e[:, 0], e[:, 1])
    packed = jnp.sort(lo * N_NODES + hi)
    valid = jnp.concatenate([jnp.ones((1,), dtype=bool),
                             packed[1:] != packed[:-1]])
    return packed // N_NODES, packed % N_NODES, valid


def _norm(x, mask=None):
    if mask is None:
        cnt = float(x.shape[0])
        mean = jnp.sum(x, axis=0) / cnt
        var = jnp.sum(x * x, axis=0) / cnt - mean * mean
    else:
        w = mask.astype(x.dtype)[:, None]
        cnt = jnp.sum(mask.astype(x.dtype))
        mean = jnp.sum(x * w, axis=0) / cnt
        var = jnp.sum((x * x) * w, axis=0) / cnt - mean * mean
    std = jnp.maximum(jnp.sqrt(jnp.maximum(var, 0.0)), 1e-8)
    return (x - mean) / std


def _prep_mlp(ps, in_dim=None):
    out = []
    for i, (w, b) in enumerate(ps):
        if i == 0 and in_dim is not None and w.shape[0] != LATENT:
            w = jnp.pad(w, ((0, LATENT - w.shape[0]), (0, 0)))
        if w.shape[1] != LATENT:
            w = jnp.pad(w, ((0, 0), (0, LATENT - w.shape[1])))
            b = jnp.pad(b, (0, LATENT - b.shape[0]))
        out.append(w.astype(jnp.float32))
        out.append(b.reshape(1, -1).astype(jnp.float32))
    return out


def kernel(world_pos, mesh_pos, node_type, cells, params_net, is_training):
    s_half, r_half, valid = _tri_edges(cells)

    one_hot = jax.nn.one_hot(node_type[:, 0], NODE_TYPE_SIZE,
                             dtype=jnp.float32)
    dwp = jnp.take(world_pos, s_half, axis=0) - jnp.take(world_pos, r_half,
                                                         axis=0)
    dmp = jnp.take(mesh_pos, s_half, axis=0) - jnp.take(mesh_pos, r_half,
                                                        axis=0)
    nwp = jnp.linalg.norm(dwp, axis=-1, keepdims=True)
    nmp = jnp.linalg.norm(dmp, axis=-1, keepdims=True)
    ef = jnp.concatenate([
        jnp.concatenate([dwp, nwp, dmp, nmp], axis=-1),
        jnp.concatenate([-dwp, nwp, -dmp, nmp], axis=-1)], axis=0)
    edge_mask = jnp.concatenate([valid, valid])
    ef = _norm(ef, mask=edge_mask)
    nf = _norm(one_hot)

    nf_pad = jnp.zeros((N_PAD, LATENT), jnp.float32)
    nf_pad = nf_pad.at[:N_NODES, :NODE_TYPE_SIZE].set(nf)
    ef_pad = jnp.zeros((2, H_PAD, LATENT), jnp.float32)
    ef_pad = ef_pad.at[0, :H, :8].set(ef[:H])
    ef_pad = ef_pad.at[1, :H, :8].set(ef[H:])

    pad_h = H_PAD - H
    s_pad = jnp.pad(s_half, (0, pad_h)).astype(jnp.int32)
    r_pad = jnp.pad(r_half, (0, pad_h)).astype(jnp.int32)
    gidx = jnp.concatenate([s_pad, r_pad]).reshape(NW, GCH, 120)
    rs = jnp.concatenate([r_pad, s_pad])
    scidx = jnp.stack([
        jnp.where(rs < HALF_N, rs, TRASH).reshape(16, SCH, 120),
        jnp.where(rs >= HALF_N, rs - HALF_N, TRASH).reshape(16, SCH, 120),
    ]).astype(jnp.int32)
    mask_h = jnp.pad(valid.astype(jnp.float32), (0, pad_h)).reshape(H_PAD, 1)

    n_enc = _prep_mlp(params_net['node_enc']['mlp'], in_dim=NODE_TYPE_SIZE)
    e_enc = _prep_mlp(params_net['edge_enc']['mlp'], in_dim=8)
    ng, nb = params_net['node_enc']['ln']
    eg, eb = params_net['edge_enc']['ln']
    node = _enc(nf_pad, *n_enc, ng.reshape(1, -1), nb.reshape(1, -1), BN)
    edge = _enc(ef_pad.reshape(E_PAD, LATENT), *e_enc, eg.reshape(1, -1),
                eb.reshape(1, -1), BE).reshape(2, H_PAD, LATENT)

    for step in params_net['steps']:
        (ew1, eb1), (ew2, eb2), (ew3, eb3) = step['edge_mlp']
        (nw1, nb1), (nw2, nb2), (nw3, nb3) = step['node_mlp']
        eg_, eb_ = step['edge_ln']
        ng_, nb_ = step['node_ln']

        u = _sc_gather(node, gidx).reshape(2, H_PAD, LATENT)
        edge, y = _edge_step(
            u, edge, mask_h,
            ew1[:LATENT], ew1[LATENT:2 * LATENT], ew1[2 * LATENT:],
            eb1.reshape(1, -1), ew2, eb2.reshape(1, -1), ew3,
            eb3.reshape(1, -1), eg_.reshape(1, -1), eb_.reshape(1, -1))
        agg = _sc_scatter(y.reshape(E_PAD, LATENT), scidx).reshape(
            N_PAD, LATENT)
        node = _node_step(
            node, agg, nw1[:LATENT], nw1[LATENT:], nb1.reshape(1, -1),
            nw2, nb2.reshape(1, -1), nw3, nb3.reshape(1, -1),
            ng_.reshape(1, -1), nb_.reshape(1, -1))

    d = _prep_mlp(params_net['dec'])
    out = _dec(node, *d, BN)[:N_NODES, :3]
    return out * jnp.asarray(is_training, dtype=out.dtype)


# speedup vs baseline: 1.1274x; 1.1274x over previous
import functools

import jax
import jax.numpy as jnp
from jax import lax
from jax.experimental import pallas as pl
from jax.experimental.pallas import tpu as pltpu
from jax.experimental.pallas import tpu_sc as plsc

N_NODES = 10000
N_CELLS = 10000
NODE_TYPE_SIZE = 9
LATENT = 128
MP_STEPS = 15

N_PAD = 10240
H = 3 * N_CELLS
H_PAD = 30720
E_PAD = 2 * H_PAD
NW = 32
NSLOT = 8

GPW = E_PAD // NW
GCH = GPW // 120
SPW = E_PAD // NW
SCH = SPW // 128
NSLOT_S = 2
ZPT = N_PAD // 16

BN = 1024
BE = 1024


def _dot(a, w):
    return lax.dot_general(a, w, (((1,), (0,)), ((), ())),
                           preferred_element_type=jnp.float32)


def _layernorm(h, g, b):
    m = jnp.mean(h, axis=-1, keepdims=True)
    v = jnp.mean((h - m) ** 2, axis=-1, keepdims=True)
    return (h - m) * lax.rsqrt(v + 1e-5) * g + b


def _enc_body(x_ref, w1, b1, w2, b2, w3, b3, g, bl, o_ref):
    h = jnp.maximum(_dot(x_ref[...], w1[...]) + b1[...], 0.0)
    h = jnp.maximum(_dot(h, w2[...]) + b2[...], 0.0)
    h = _dot(h, w3[...]) + b3[...]
    o_ref[...] = _layernorm(h, g[...], bl[...])


def _dec_body(x_ref, w1, b1, w2, b2, w3, b3, o_ref):
    h = jnp.maximum(_dot(x_ref[...], w1[...]) + b1[...], 0.0)
    h = jnp.maximum(_dot(h, w2[...]) + b2[...], 0.0)
    o_ref[...] = _dot(h, w3[...]) + b3[...]


def _edge_body(ua, ub, e, mk, w1a, w1b, w1c, b1, w2, b2, w3, b3, g, bl,
               ne_ref, y_ref):
    a = ua[0]
    b = ub[0]
    e0 = e[0]
    h = (_dot(a, w1a[...]) + _dot(b, w1b[...]) + _dot(e0, w1c[...]) + b1[...])
    h = jnp.maximum(h, 0.0)
    h = jnp.maximum(_dot(h, w2[...]) + b2[...], 0.0)
    h = _dot(h, w3[...]) + b3[...]
    ne = _layernorm(h, g[...], bl[...]) + e0
    ne_ref[0] = ne
    y_ref[0] = ne * mk[...]


def _node_body(nd, p0, p1, w1a, w1b, b1, w2, b2, w3, b3, g, bl, o_ref):
    ag = p0[0] + p1[0]
    h = jnp.maximum(_dot(nd[...], w1a[...]) + _dot(ag, w1b[...])
                    + b1[...], 0.0)
    h = jnp.maximum(_dot(h, w2[...]) + b2[...], 0.0)
    h = _dot(h, w3[...]) + b3[...]
    o_ref[...] = _layernorm(h, g[...], bl[...]) + nd[...]


def _rowspec(blk):
    return pl.BlockSpec((blk, LATENT), lambda i: (i, 0))


_WSPEC = pl.BlockSpec((LATENT, LATENT), lambda i: (0, 0))
_VSPEC = pl.BlockSpec((1, LATENT), lambda i: (0, 0))
_WSPEC2 = pl.BlockSpec((LATENT, LATENT), lambda h, i: (0, 0))
_VSPEC2 = pl.BlockSpec((1, LATENT), lambda h, i: (0, 0))


def _enc(x, w1, b1, w2, b2, w3, b3, g, bl, blk):
    rows = x.shape[0]
    return pl.pallas_call(
        _enc_body,
        grid=(rows // blk,),
        in_specs=[_rowspec(blk)] + [_WSPEC, _VSPEC] * 3 + [_VSPEC, _VSPEC],
        out_specs=_rowspec(blk),
        out_shape=jax.ShapeDtypeStruct((rows, LATENT), jnp.float32),
    )(x, w1, b1, w2, b2, w3, b3, g, bl)


def _dec(x, w1, b1, w2, b2, w3, b3, blk):
    rows = x.shape[0]
    return pl.pallas_call(
        _dec_body,
        grid=(rows // blk,),
        in_specs=[_rowspec(blk)] + [_WSPEC, _VSPEC] * 3,
        out_specs=_rowspec(blk),
        out_shape=jax.ShapeDtypeStruct((rows, LATENT), jnp.float32),
    )(x, w1, b1, w2, b2, w3, b3)


def _edge_step(u, edge3, mk, w1a, w1b, w1c, b1, w2, b2, w3, b3, g, bl):
    half = pl.BlockSpec((1, BE, LATENT), lambda h, i: (h, i, 0))
    swap = pl.BlockSpec((1, BE, LATENT), lambda h, i: (1 - h, i, 0))
    mspec = pl.BlockSpec((BE, 1), lambda h, i: (i, 0))
    return pl.pallas_call(
        _edge_body,
        grid=(2, H_PAD // BE),
        in_specs=[half, swap, half, mspec,
                  _WSPEC2, _WSPEC2, _WSPEC2, _VSPEC2,
                  _WSPEC2, _VSPEC2, _WSPEC2, _VSPEC2, _VSPEC2, _VSPEC2],
        out_specs=[half, half],
        out_shape=[jax.ShapeDtypeStruct((2, H_PAD, LATENT), jnp.float32),
                   jax.ShapeDtypeStruct((2, H_PAD, LATENT), jnp.float32)],
    )(u, u, edge3, mk, w1a, w1b, w1c, b1, w2, b2, w3, b3, g, bl)


def _node_step(node, parts, w1a, w1b, b1, w2, b2, w3, b3, g, bl):
    p0 = pl.BlockSpec((1, BN, LATENT), lambda i: (0, i, 0))
    p1 = pl.BlockSpec((1, BN, LATENT), lambda i: (1, i, 0))
    return pl.pallas_call(
        _node_body,
        grid=(N_PAD // BN,),
        in_specs=[_rowspec(BN), p0, p1,
                  _WSPEC, _WSPEC, _VSPEC,
                  _WSPEC, _VSPEC, _WSPEC, _VSPEC, _VSPEC, _VSPEC],
        out_specs=_rowspec(BN),
        out_shape=jax.ShapeDtypeStruct((N_PAD, LATENT), jnp.float32),
    )(node, parts, parts, w1a, w1b, b1, w2, b2, w3, b3, g, bl)


@functools.cache
def _build_sc_gather():
    mesh = plsc.VectorSubcoreMesh(core_axis_name="c", subcore_axis_name="s")

    @functools.partial(
        pl.kernel,
        mesh=mesh,
        out_type=jax.ShapeDtypeStruct((E_PAD, LATENT), jnp.float32),
        scratch_types=[pltpu.VMEM((GCH, 120), jnp.int32)]
        + [pltpu.VMEM((120, LATENT), jnp.float32)] * NSLOT
        + [pltpu.SemaphoreType.DMA] * (2 * NSLOT),
    )
    def gather_kernel(node_hbm, idx_hbm, out_hbm, idx_v, *rest):
        bufs = rest[:NSLOT]
        gsem = rest[NSLOT:2 * NSLOT]
        wsem = rest[2 * NSLOT:]
        wid = lax.axis_index("s") * 2 + lax.axis_index("c")
        pltpu.sync_copy(idx_hbm.at[wid], idx_v)
        base = wid * GPW

        def _out_at(c):
            return out_hbm.at[pl.ds(base + c * 120, 120)]

        for k in range(NSLOT):
            pltpu.async_copy(node_hbm.at[idx_v.at[k]], bufs[k], gsem[k])

        def cycle(t, carry):
            for k in range(NSLOT):
                c = t * NSLOT + k
                pltpu.make_async_copy(node_hbm.at[idx_v.at[c]], bufs[k],
                                      gsem[k]).wait()
                pltpu.async_copy(bufs[k], _out_at(c), wsem[k])
            for k in range(NSLOT):
                c = t * NSLOT + k
                n = c + NSLOT

                @pl.when(n < GCH)
                def _():
                    pltpu.make_async_copy(bufs[k], _out_at(c), wsem[k]).wait()
                    pltpu.async_copy(node_hbm.at[idx_v.at[n]], bufs[k],
                                     gsem[k])
            return carry

        lax.fori_loop(0, GCH // NSLOT, cycle, 0)
        for k in range(NSLOT):
            pltpu.make_async_copy(bufs[k], _out_at(GCH - NSLOT + k),
                                  wsem[k]).wait()

    return gather_kernel


def _sc_gather(node, gidx):
    return _build_sc_gather()(node, gidx)


@functools.cache
def _build_sc_scatter():
    mesh = plsc.VectorSubcoreMesh(core_axis_name="c", subcore_axis_name="s")

    @functools.partial(
        pl.kernel,
        mesh=mesh,
        out_type=jax.ShapeDtypeStruct((2, N_PAD, LATENT), jnp.float32),
        scratch_types=[pltpu.VMEM((SCH, 128), jnp.int32),
                       pltpu.VMEM_SHARED((N_PAD, LATENT), jnp.float32)]
        + [pltpu.VMEM((128, LATENT), jnp.float32)] * NSLOT_S
        + [pltpu.SemaphoreType.DMA] * (2 * NSLOT_S),
    )
    def scatter_kernel(y_hbm, idx_hbm, out_hbm, idx_v, shared, *rest):
        bufs = rest[:NSLOT_S]
        lsem = rest[NSLOT_S:2 * NSLOT_S]
        ssem = rest[2 * NSLOT_S:]
        cc = lax.axis_index("c")
        s = lax.axis_index("s")
        wid = s * 2 + cc

        def zrow(i, carry):
            for j in range(LATENT // 16):
                bufs[0][i, pl.ds(j * 16, 16)] = jnp.zeros((16,), jnp.float32)
            return carry

        lax.fori_loop(0, 128, zrow, 0)
        z0 = s * ZPT
        zcp = [pltpu.async_copy(bufs[0], shared.at[pl.ds(z0 + i * 128, 128)],
                                lsem[i % NSLOT_S])
               for i in range(ZPT // 128)]
        for d in zcp:
            d.wait()
        plsc.subcore_barrier()

        pltpu.sync_copy(idx_hbm.at[wid], idx_v)
        base = wid * SPW

        def _y_at(c):
            return y_hbm.at[pl.ds(base + c * 128, 128)]

        for k in range(NSLOT_S):
            pltpu.async_copy(_y_at(k), bufs[k], lsem[k])

        def cycle(t, carry):
            for k in range(NSLOT_S):
                c = t * NSLOT_S + k

                @pl.when(c < SCH)
                def _():
                    pltpu.make_async_copy(_y_at(c), bufs[k], lsem[k]).wait()
                    pltpu.async_copy(bufs[k], shared.at[idx_v.at[c]],
                                     ssem[k], add=True)
            for k in range(NSLOT_S):
                c = t * NSLOT_S + k
                n = c + NSLOT_S

                @pl.when(n < SCH)
                def _():
                    pltpu.make_async_copy(bufs[k], shared.at[idx_v.at[c]],
                                          ssem[k]).wait()
                    pltpu.async_copy(_y_at(n), bufs[k], lsem[k])
            return carry

        lax.fori_loop(0, (SCH + NSLOT_S - 1) // NSLOT_S, cycle, 0)
        for k in range(NSLOT_S):
            c = SCH - NSLOT_S + k
            pltpu.make_async_copy(bufs[k], shared.at[idx_v.at[c]],
                                  ssem[k]).wait()
        plsc.subcore_barrier()

        ocp = [None] * NSLOT_S
        for i in range(ZPT // 128):
            k = i % NSLOT_S
            if ocp[k] is not None:
                ocp[k].wait()
            pltpu.async_copy(shared.at[pl.ds(z0 + i * 128, 128)],
                             bufs[k], lsem[k]).wait()
            ocp[k] = pltpu.async_copy(
                bufs[k], out_hbm.at[cc, pl.ds(z0 + i * 128, 128)], ssem[k])
        for d in ocp:
            if d is not None:
                d.wait()

    return scatter_kernel


def _sc_scatter(y, sidx):
    return _build_sc_scatter()(y, sidx)


def _tri_edges(cells):
    cells = cells.astype(jnp.int32)
    e = jnp.concatenate([cells[:, 0:2], cells[:, 1:3],
                         jnp.stack([cells[:, 2], cells[:, 0]], axis=1)],
                        axis=0)
    lo = jnp.minimum(e[:, 0], e[:, 1])
    hi = jnp.maximum(e[:, 0], e[:, 1])
    packed = jnp.sort(lo * N_NODES + hi)
    valid = jnp.concatenate([jnp.ones((1,), dtype=bool),
                             packed[1:] != packed[:-1]])
    return packed // N_NODES, packed % N_NODES, valid


def _norm(x, mask=None):
    if mask is None:
        cnt = float(x.shape[0])
        mean = jnp.sum(x, axis=0) / cnt
        var = jnp.sum(x * x, axis=0) / cnt - mean * mean
    else:
        w = mask.astype(x.dtype)[:, None]
        cnt = jnp.sum(mask.astype(x.dtype))
        mean = jnp.sum(x * w, axis=0) / cnt
        var = jnp.sum((x * x) * w, axis=0) / cnt - mean * mean
    std = jnp.maximum(jnp.sqrt(jnp.maximum(var, 0.0)), 1e-8)
    return (x - mean) / std


def _prep_mlp(ps, in_dim=None):
    out = []
    for i, (w, b) in enumerate(ps):
        if i == 0 and in_dim is not None and w.shape[0] != LATENT:
            w = jnp.pad(w, ((0, LATENT - w.shape[0]), (0, 0)))
        if w.shape[1] != LATENT:
            w = jnp.pad(w, ((0, 0), (0, LATENT - w.shape[1])))
            b = jnp.pad(b, (0, LATENT - b.shape[0]))
        out.append(w.astype(jnp.float32))
        out.append(b.reshape(1, -1).astype(jnp.float32))
    return out


def kernel(world_pos, mesh_pos, node_type, cells, params_net, is_training):
    s_half, r_half, valid = _tri_edges(cells)

    one_hot = jax.nn.one_hot(node_type[:, 0], NODE_TYPE_SIZE,
                             dtype=jnp.float32)
    dwp = jnp.take(world_pos, s_half, axis=0) - jnp.take(world_pos, r_half,
                                                         axis=0)
    dmp = jnp.take(mesh_pos, s_half, axis=0) - jnp.take(mesh_pos, r_half,
                                                        axis=0)
    nwp = jnp.linalg.norm(dwp, axis=-1, keepdims=True)
    nmp = jnp.linalg.norm(dmp, axis=-1, keepdims=True)
    ef = jnp.concatenate([
        jnp.concatenate([dwp, nwp, dmp, nmp], axis=-1),
        jnp.concatenate([-dwp, nwp, -dmp, nmp], axis=-1)], axis=0)
    edge_mask = jnp.concatenate([valid, valid])
    ef = _norm(ef, mask=edge_mask)
    nf = _norm(one_hot)

    nf_pad = jnp.zeros((N_PAD, LATENT), jnp.float32)
    nf_pad = nf_pad.at[:N_NODES, :NODE_TYPE_SIZE].set(nf)
    ef_pad = jnp.zeros((2, H_PAD, LATENT), jnp.float32)
    ef_pad = ef_pad.at[0, :H, :8].set(ef[:H])
    ef_pad = ef_pad.at[1, :H, :8].set(ef[H:])

    pad_h = H_PAD - H
    s_pad = jnp.pad(s_half, (0, pad_h)).astype(jnp.int32)
    r_pad = jnp.pad(r_half, (0, pad_h)).astype(jnp.int32)
    gidx = jnp.concatenate([s_pad, r_pad]).reshape(NW, GCH, 120)
    scidx = jnp.concatenate([r_pad, s_pad]).reshape(NW, SCH, 128)
    mask_h = jnp.pad(valid.astype(jnp.float32), (0, pad_h)).reshape(H_PAD, 1)

    n_enc = _prep_mlp(params_net['node_enc']['mlp'], in_dim=NODE_TYPE_SIZE)
    e_enc = _prep_mlp(params_net['edge_enc']['mlp'], in_dim=8)
    ng, nb = params_net['node_enc']['ln']
    eg, eb = params_net['edge_enc']['ln']
    node = _enc(nf_pad, *n_enc, ng.reshape(1, -1), nb.reshape(1, -1), BN)
    edge = _enc(ef_pad.reshape(E_PAD, LATENT), *e_enc, eg.reshape(1, -1),
                eb.reshape(1, -1), BE).reshape(2, H_PAD, LATENT)

    for step in params_net['steps']:
        (ew1, eb1), (ew2, eb2), (ew3, eb3) = step['edge_mlp']
        (nw1, nb1), (nw2, nb2), (nw3, nb3) = step['node_mlp']
        eg_, eb_ = step['edge_ln']
        ng_, nb_ = step['node_ln']

        u = _sc_gather(node, gidx).reshape(2, H_PAD, LATENT)
        edge, y = _edge_step(
            u, edge, mask_h,
            ew1[:LATENT], ew1[LATENT:2 * LATENT], ew1[2 * LATENT:],
            eb1.reshape(1, -1), ew2, eb2.reshape(1, -1), ew3,
            eb3.reshape(1, -1), eg_.reshape(1, -1), eb_.reshape(1, -1))
        parts = _sc_scatter(y.reshape(E_PAD, LATENT), scidx)
        node = _node_step(
            node, parts, nw1[:LATENT], nw1[LATENT:], nb1.reshape(1, -1),
            nw2, nb2.reshape(1, -1), nw3, nb3.reshape(1, -1),
            ng_.reshape(1, -1), nb_.reshape(1, -1))

    d = _prep_mlp(params_net['dec'])
    out = _dec(node, *d, BN)[:N_NODES, :3]
    return out * jnp.asarray(is_training, dtype=out.dtype)


# speedup vs baseline: 1.2548x; 1.1130x over previous
import functools

import jax
import jax.numpy as jnp
from jax import lax
from jax.experimental import pallas as pl
from jax.experimental.pallas import tpu as pltpu
from jax.experimental.pallas import tpu_sc as plsc

N_NODES = 10000
N_CELLS = 10000
NODE_TYPE_SIZE = 9
LATENT = 128
MP_STEPS = 15

N_PAD = 10240
H = 3 * N_CELLS
H_PAD = 30720
E_PAD = 2 * H_PAD
NW = 32
NSLOT = 8

GPW = E_PAD // NW
GCH = GPW // 120
SPW = E_PAD // NW
SCH = SPW // 120
NSLOT_S = 3
ZPT = N_PAD // 16

BN = 2048
BE = 2048


def _dot(a, w):
    return lax.dot_general(a, w, (((1,), (0,)), ((), ())),
                           preferred_element_type=jnp.float32)


def _layernorm(h, g, b):
    m = jnp.mean(h, axis=-1, keepdims=True)
    v = jnp.mean((h - m) ** 2, axis=-1, keepdims=True)
    return (h - m) * lax.rsqrt(v + 1e-5) * g + b


def _enc_body(x_ref, w1, b1, w2, b2, w3, b3, g, bl, o_ref):
    h = jnp.maximum(_dot(x_ref[...], w1[...]) + b1[...], 0.0)
    h = jnp.maximum(_dot(h, w2[...]) + b2[...], 0.0)
    h = _dot(h, w3[...]) + b3[...]
    o_ref[...] = _layernorm(h, g[...], bl[...])


def _dec_body(x_ref, w1, b1, w2, b2, w3, b3, o_ref):
    h = jnp.maximum(_dot(x_ref[...], w1[...]) + b1[...], 0.0)
    h = jnp.maximum(_dot(h, w2[...]) + b2[...], 0.0)
    o_ref[...] = _dot(h, w3[...]) + b3[...]


def _edge_body(ua, ub, e, mk, w1a, w1b, w1c, b1, w2, b2, w3, b3, g, bl,
               ne_ref, y_ref):
    a = ua[0]
    b = ub[0]
    e0 = e[0]
    h = (_dot(a, w1a[...]) + _dot(b, w1b[...]) + _dot(e0, w1c[...]) + b1[...])
    h = jnp.maximum(h, 0.0)
    h = jnp.maximum(_dot(h, w2[...]) + b2[...], 0.0)
    h = _dot(h, w3[...]) + b3[...]
    ne = _layernorm(h, g[...], bl[...]) + e0
    ne_ref[0] = ne
    y_ref[0] = ne * mk[...]


def _node_body(nd, p0, p1, w1a, w1b, b1, w2, b2, w3, b3, g, bl, o_ref):
    ag = p0[0] + p1[0]
    h = jnp.maximum(_dot(nd[...], w1a[...]) + _dot(ag, w1b[...])
                    + b1[...], 0.0)
    h = jnp.maximum(_dot(h, w2[...]) + b2[...], 0.0)
    h = _dot(h, w3[...]) + b3[...]
    o_ref[...] = _layernorm(h, g[...], bl[...]) + nd[...]


def _rowspec(blk):
    return pl.BlockSpec((blk, LATENT), lambda i: (i, 0))


_WSPEC = pl.BlockSpec((LATENT, LATENT), lambda i: (0, 0))
_VSPEC = pl.BlockSpec((1, LATENT), lambda i: (0, 0))
_WSPEC2 = pl.BlockSpec((LATENT, LATENT), lambda h, i: (0, 0))
_VSPEC2 = pl.BlockSpec((1, LATENT), lambda h, i: (0, 0))


def _enc(x, w1, b1, w2, b2, w3, b3, g, bl, blk):
    rows = x.shape[0]
    return pl.pallas_call(
        _enc_body,
        grid=(rows // blk,),
        in_specs=[_rowspec(blk)] + [_WSPEC, _VSPEC] * 3 + [_VSPEC, _VSPEC],
        out_specs=_rowspec(blk),
        out_shape=jax.ShapeDtypeStruct((rows, LATENT), jnp.float32),
    )(x, w1, b1, w2, b2, w3, b3, g, bl)


def _dec(x, w1, b1, w2, b2, w3, b3, blk):
    rows = x.shape[0]
    return pl.pallas_call(
        _dec_body,
        grid=(rows // blk,),
        in_specs=[_rowspec(blk)] + [_WSPEC, _VSPEC] * 3,
        out_specs=_rowspec(blk),
        out_shape=jax.ShapeDtypeStruct((rows, LATENT), jnp.float32),
    )(x, w1, b1, w2, b2, w3, b3)


def _edge_step(u, edge3, mk, w1a, w1b, w1c, b1, w2, b2, w3, b3, g, bl):
    half = pl.BlockSpec((1, BE, LATENT), lambda h, i: (h, i, 0))
    swap = pl.BlockSpec((1, BE, LATENT), lambda h, i: (1 - h, i, 0))
    mspec = pl.BlockSpec((BE, 1), lambda h, i: (i, 0))
    return pl.pallas_call(
        _edge_body,
        grid=(2, H_PAD // BE),
        in_specs=[half, swap, half, mspec,
                  _WSPEC2, _WSPEC2, _WSPEC2, _VSPEC2,
                  _WSPEC2, _VSPEC2, _WSPEC2, _VSPEC2, _VSPEC2, _VSPEC2],
        out_specs=[half, half],
        out_shape=[jax.ShapeDtypeStruct((2, H_PAD, LATENT), jnp.float32),
                   jax.ShapeDtypeStruct((2, H_PAD, LATENT), jnp.float32)],
    )(u, u, edge3, mk, w1a, w1b, w1c, b1, w2, b2, w3, b3, g, bl)


def _node_step(node, parts, w1a, w1b, b1, w2, b2, w3, b3, g, bl):
    p0 = pl.BlockSpec((1, BN, LATENT), lambda i: (0, i, 0))
    p1 = pl.BlockSpec((1, BN, LATENT), lambda i: (1, i, 0))
    return pl.pallas_call(
        _node_body,
        grid=(N_PAD // BN,),
        in_specs=[_rowspec(BN), p0, p1,
                  _WSPEC, _WSPEC, _VSPEC,
                  _WSPEC, _VSPEC, _WSPEC, _VSPEC, _VSPEC, _VSPEC],
        out_specs=_rowspec(BN),
        out_shape=jax.ShapeDtypeStruct((N_PAD, LATENT), jnp.float32),
    )(node, parts, parts, w1a, w1b, b1, w2, b2, w3, b3, g, bl)


@functools.cache
def _build_sc_gather():
    mesh = plsc.VectorSubcoreMesh(core_axis_name="c", subcore_axis_name="s")

    @functools.partial(
        pl.kernel,
        mesh=mesh,
        out_type=jax.ShapeDtypeStruct((E_PAD, LATENT), jnp.float32),
        scratch_types=[pltpu.VMEM((GCH, 120), jnp.int32)]
        + [pltpu.VMEM((120, LATENT), jnp.float32)] * NSLOT
        + [pltpu.SemaphoreType.DMA] * (2 * NSLOT),
    )
    def gather_kernel(node_hbm, idx_hbm, out_hbm, idx_v, *rest):
        bufs = rest[:NSLOT]
        gsem = rest[NSLOT:2 * NSLOT]
        wsem = rest[2 * NSLOT:]
        wid = lax.axis_index("s") * 2 + lax.axis_index("c")
        pltpu.sync_copy(idx_hbm.at[wid], idx_v)
        base = wid * GPW

        def _out_at(c):
            return out_hbm.at[pl.ds(base + c * 120, 120)]

        for k in range(NSLOT):
            pltpu.async_copy(node_hbm.at[idx_v.at[k]], bufs[k], gsem[k])

        def cycle(t, carry):
            for k in range(NSLOT):
                c = t * NSLOT + k
                pltpu.make_async_copy(node_hbm.at[idx_v.at[c]], bufs[k],
                                      gsem[k]).wait()
                pltpu.async_copy(bufs[k], _out_at(c), wsem[k])
            for k in range(NSLOT):
                c = t * NSLOT + k
                n = c + NSLOT

                @pl.when(n < GCH)
                def _():
                    pltpu.make_async_copy(bufs[k], _out_at(c), wsem[k]).wait()
                    pltpu.async_copy(node_hbm.at[idx_v.at[n]], bufs[k],
                                     gsem[k])
            return carry

        lax.fori_loop(0, GCH // NSLOT, cycle, 0)
        for k in range(NSLOT):
            pltpu.make_async_copy(bufs[k], _out_at(GCH - NSLOT + k),
                                  wsem[k]).wait()

    return gather_kernel


def _sc_gather(node, gidx):
    return _build_sc_gather()(node, gidx)


@functools.cache
def _build_sc_scatter():
    mesh = plsc.VectorSubcoreMesh(core_axis_name="c", subcore_axis_name="s")

    @functools.partial(
        pl.kernel,
        mesh=mesh,
        out_type=jax.ShapeDtypeStruct((2, N_PAD, LATENT), jnp.float32),
        scratch_types=[pltpu.VMEM((SCH, 120), jnp.int32),
                       pltpu.VMEM_SHARED((N_PAD, LATENT), jnp.float32)]
        + [pltpu.VMEM((120, LATENT), jnp.float32)] * NSLOT_S
        + [pltpu.SemaphoreType.DMA] * (2 * NSLOT_S),
    )
    def scatter_kernel(y_hbm, idx_hbm, out_hbm, idx_v, shared, *rest):
        bufs = rest[:NSLOT_S]
        lsem = rest[NSLOT_S:2 * NSLOT_S]
        ssem = rest[2 * NSLOT_S:]
        cc = lax.axis_index("c")
        s = lax.axis_index("s")
        wid = s * 2 + cc

        def zrow(i, carry):
            for j in range(LATENT // 16):
                bufs[0][i, pl.ds(j * 16, 16)] = jnp.zeros((16,), jnp.float32)
            return carry

        lax.fori_loop(0, 120, zrow, 0)
        z0 = s * ZPT
        zcp = [pltpu.async_copy(bufs[0], shared.at[pl.ds(z0 + off, 120)],
                                lsem[i % NSLOT_S])
               for i, off in enumerate((0, 120, 240, 360, 480, 520))]
        for d in zcp:
            d.wait()
        plsc.subcore_barrier()

        pltpu.sync_copy(idx_hbm.at[wid], idx_v)
        base = wid * SPW

        def _y_at(c):
            return y_hbm.at[pl.ds(base + c * 120, 120)]

        for k in range(NSLOT_S):
            pltpu.async_copy(_y_at(k), bufs[k], lsem[k])

        def cycle(t, carry):
            for k in range(NSLOT_S):
                c = t * NSLOT_S + k

                @pl.when(c < SCH)
                def _():
                    pltpu.make_async_copy(_y_at(c), bufs[k], lsem[k]).wait()
                    pltpu.async_copy(bufs[k], shared.at[idx_v.at[c]],
                                     ssem[k], add=True)
            for k in range(NSLOT_S):
                c = t * NSLOT_S + k
                n = c + NSLOT_S

                @pl.when(n < SCH)
                def _():
                    pltpu.make_async_copy(bufs[k], shared.at[idx_v.at[c]],
                                          ssem[k]).wait()
                    pltpu.async_copy(_y_at(n), bufs[k], lsem[k])
            return carry

        lax.fori_loop(0, (SCH + NSLOT_S - 1) // NSLOT_S, cycle, 0)
        for k in range(NSLOT_S):
            c = SCH - NSLOT_S + k
            pltpu.make_async_copy(bufs[k], shared.at[idx_v.at[c]],
                                  ssem[k]).wait()
        plsc.subcore_barrier()

        ocp = [None] * NSLOT_S
        for i, off in enumerate((0, 120, 240, 360, 480, 520)):
            k = i % NSLOT_S
            if ocp[k] is not None:
                ocp[k].wait()
            pltpu.async_copy(shared.at[pl.ds(z0 + off, 120)],
                             bufs[k], lsem[k]).wait()
            ocp[k] = pltpu.async_copy(
                bufs[k], out_hbm.at[cc, pl.ds(z0 + off, 120)], ssem[k])
        for d in ocp:
            if d is not None:
                d.wait()

    return scatter_kernel


def _sc_scatter(y, sidx):
    return _build_sc_scatter()(y, sidx)


def _tri_edges(cells):
    cells = cells.astype(jnp.int32)
    e = jnp.concatenate([cells[:, 0:2], cells[:, 1:3],
                         jnp.stack([cells[:, 2], cells[:, 0]], axis=1)],
                        axis=0)
    lo = jnp.minimum(e[:, 0], e[:, 1])
    hi = jnp.maximum(e[:, 0], e[:, 1])
    packed = jnp.sort(lo * N_NODES + hi)
    valid = jnp.concatenate([jnp.ones((1,), dtype=bool),
                             packed[1:] != packed[:-1]])
    return packed // N_NODES, packed % N_NODES, valid


def _norm(x, mask=None):
    if mask is None:
        cnt = float(x.shape[0])
        mean = jnp.sum(x, axis=0) / cnt
        var = jnp.sum(x * x, axis=0) / cnt - mean * mean
    else:
        w = mask.astype(x.dtype)[:, None]
        cnt = jnp.sum(mask.astype(x.dtype))
        mean = jnp.sum(x * w, axis=0) / cnt
        var = jnp.sum((x * x) * w, axis=0) / cnt - mean * mean
    std = jnp.maximum(jnp.sqrt(jnp.maximum(var, 0.0)), 1e-8)
    return (x - mean) / std


def _prep_mlp(ps, in_dim=None):
    out = []
    for i, (w, b) in enumerate(ps):
        if i == 0 and in_dim is not None and w.shape[0] != LATENT:
            w = jnp.pad(w, ((0, LATENT - w.shape[0]), (0, 0)))
        if w.shape[1] != LATENT:
            w = jnp.pad(w, ((0, 0), (0, LATENT - w.shape[1])))
            b = jnp.pad(b, (0, LATENT - b.shape[0]))
        out.append(w.astype(jnp.float32))
        out.append(b.reshape(1, -1).astype(jnp.float32))
    return out


def kernel(world_pos, mesh_pos, node_type, cells, params_net, is_training):
    s_half, r_half, valid = _tri_edges(cells)

    one_hot = jax.nn.one_hot(node_type[:, 0], NODE_TYPE_SIZE,
                             dtype=jnp.float32)
    dwp = jnp.take(world_pos, s_half, axis=0) - jnp.take(world_pos, r_half,
                                                         axis=0)
    dmp = jnp.take(mesh_pos, s_half, axis=0) - jnp.take(mesh_pos, r_half,
                                                        axis=0)
    nwp = jnp.linalg.norm(dwp, axis=-1, keepdims=True)
    nmp = jnp.linalg.norm(dmp, axis=-1, keepdims=True)
    ef = jnp.concatenate([
        jnp.concatenate([dwp, nwp, dmp, nmp], axis=-1),
        jnp.concatenate([-dwp, nwp, -dmp, nmp], axis=-1)], axis=0)
    edge_mask = jnp.concatenate([valid, valid])
    ef = _norm(ef, mask=edge_mask)
    nf = _norm(one_hot)

    nf_pad = jnp.zeros((N_PAD, LATENT), jnp.float32)
    nf_pad = nf_pad.at[:N_NODES, :NODE_TYPE_SIZE].set(nf)
    ef_pad = jnp.zeros((2, H_PAD, LATENT), jnp.float32)
    ef_pad = ef_pad.at[0, :H, :8].set(ef[:H])
    ef_pad = ef_pad.at[1, :H, :8].set(ef[H:])

    pad_h = H_PAD - H
    s_pad = jnp.pad(s_half, (0, pad_h)).astype(jnp.int32)
    r_pad = jnp.pad(r_half, (0, pad_h)).astype(jnp.int32)
    gidx = jnp.concatenate([s_pad, r_pad]).reshape(NW, GCH, 120)
    scidx = jnp.concatenate([r_pad, s_pad]).reshape(NW, SCH, 120)
    mask_h = jnp.pad(valid.astype(jnp.float32), (0, pad_h)).reshape(H_PAD, 1)

    n_enc = _prep_mlp(params_net['node_enc']['mlp'], in_dim=NODE_TYPE_SIZE)
    e_enc = _prep_mlp(params_net['edge_enc']['mlp'], in_dim=8)
    ng, nb = params_net['node_enc']['ln']
    eg, eb = params_net['edge_enc']['ln']
    node = _enc(nf_pad, *n_enc, ng.reshape(1, -1), nb.reshape(1, -1), BN)
    edge = _enc(ef_pad.reshape(E_PAD, LATENT), *e_enc, eg.reshape(1, -1),
                eb.reshape(1, -1), BE).reshape(2, H_PAD, LATENT)

    for step in params_net['steps']:
        (ew1, eb1), (ew2, eb2), (ew3, eb3) = step['edge_mlp']
        (nw1, nb1), (nw2, nb2), (nw3, nb3) = step['node_mlp']
        eg_, eb_ = step['edge_ln']
        ng_, nb_ = step['node_ln']

        u = _sc_gather(node, gidx).reshape(2, H_PAD, LATENT)
        edge, y = _edge_step(
            u, edge, mask_h,
            ew1[:LATENT], ew1[LATENT:2 * LATENT], ew1[2 * LATENT:],
            eb1.reshape(1, -1), ew2, eb2.reshape(1, -1), ew3,
            eb3.reshape(1, -1), eg_.reshape(1, -1), eb_.reshape(1, -1))
        parts = _sc_scatter(y.reshape(E_PAD, LATENT), scidx)
        node = _node_step(
            node, parts, nw1[:LATENT], nw1[LATENT:], nb1.reshape(1, -1),
            nw2, nb2.reshape(1, -1), nw3, nb3.reshape(1, -1),
            ng_.reshape(1, -1), nb_.reshape(1, -1))

    d = _prep_mlp(params_net['dec'])
    out = _dec(node, *d, BN)[:N_NODES, :3]
    return out * jnp.asarray(is_training, dtype=out.dtype)


# speedup vs baseline: 1.2845x; 1.0236x over previous
import functools

import jax
import jax.numpy as jnp
from jax import lax
from jax.experimental import pallas as pl
from jax.experimental.pallas import tpu as pltpu
from jax.experimental.pallas import tpu_sc as plsc

N_NODES = 10000
N_CELLS = 10000
NODE_TYPE_SIZE = 9
LATENT = 128
MP_STEPS = 15

N_PAD = 10240
H = 3 * N_CELLS
H_PAD = 30720
E_PAD = 2 * H_PAD
NW = 32
NSLOT = 8

GPW = E_PAD // NW
GCH = GPW // 120
SPW = E_PAD // NW
SCH = SPW // 120
NSLOT_S = 3
ZPT = N_PAD // 16

BN = 2048
BE = 3072


def _dot(a, w):
    return lax.dot_general(a, w, (((1,), (0,)), ((), ())),
                           preferred_element_type=jnp.float32)


def _layernorm(h, g, b):
    m = jnp.mean(h, axis=-1, keepdims=True)
    v = jnp.mean((h - m) ** 2, axis=-1, keepdims=True)
    return (h - m) * lax.rsqrt(v + 1e-5) * g + b


def _enc_body(x_ref, w1, b1, w2, b2, w3, b3, g, bl, o_ref):
    h = jnp.maximum(_dot(x_ref[...], w1[...]) + b1[...], 0.0)
    h = jnp.maximum(_dot(h, w2[...]) + b2[...], 0.0)
    h = _dot(h, w3[...]) + b3[...]
    o_ref[...] = _layernorm(h, g[...], bl[...])


def _dec_body(x_ref, w1, b1, w2, b2, w3, b3, o_ref):
    h = jnp.maximum(_dot(x_ref[...], w1[...]) + b1[...], 0.0)
    h = jnp.maximum(_dot(h, w2[...]) + b2[...], 0.0)
    o_ref[...] = _dot(h, w3[...]) + b3[...]


def _edge_body(ua, ub, e, mk, w1a, w1b, w1c, b1, w2, b2, w3, b3, g, bl,
               ne_ref, y_ref):
    a = ua[0]
    b = ub[0]
    e0 = e[0]
    h = (_dot(a, w1a[...]) + _dot(b, w1b[...]) + _dot(e0, w1c[...]) + b1[...])
    h = jnp.maximum(h, 0.0)
    h = jnp.maximum(_dot(h, w2[...]) + b2[...], 0.0)
    h = _dot(h, w3[...]) + b3[...]
    ne = _layernorm(h, g[...], bl[...]) + e0
    ne_ref[0] = ne
    y_ref[0] = ne * mk[...]


def _node_body(nd, p0, p1, w1a, w1b, b1, w2, b2, w3, b3, g, bl, o_ref):
    ag = p0[0] + p1[0]
    h = jnp.maximum(_dot(nd[...], w1a[...]) + _dot(ag, w1b[...])
                    + b1[...], 0.0)
    h = jnp.maximum(_dot(h, w2[...]) + b2[...], 0.0)
    h = _dot(h, w3[...]) + b3[...]
    o_ref[...] = _layernorm(h, g[...], bl[...]) + nd[...]


def _rowspec(blk):
    return pl.BlockSpec((blk, LATENT), lambda i: (i, 0))


_WSPEC = pl.BlockSpec((LATENT, LATENT), lambda i: (0, 0))
_VSPEC = pl.BlockSpec((1, LATENT), lambda i: (0, 0))
_WSPEC2 = pl.BlockSpec((LATENT, LATENT), lambda h, i: (0, 0))
_VSPEC2 = pl.BlockSpec((1, LATENT), lambda h, i: (0, 0))


def _enc(x, w1, b1, w2, b2, w3, b3, g, bl, blk):
    rows = x.shape[0]
    return pl.pallas_call(
        _enc_body,
        grid=(rows // blk,),
        in_specs=[_rowspec(blk)] + [_WSPEC, _VSPEC] * 3 + [_VSPEC, _VSPEC],
        out_specs=_rowspec(blk),
        out_shape=jax.ShapeDtypeStruct((rows, LATENT), jnp.float32),
    )(x, w1, b1, w2, b2, w3, b3, g, bl)


def _dec(x, w1, b1, w2, b2, w3, b3, blk):
    rows = x.shape[0]
    return pl.pallas_call(
        _dec_body,
        grid=(rows // blk,),
        in_specs=[_rowspec(blk)] + [_WSPEC, _VSPEC] * 3,
        out_specs=_rowspec(blk),
        out_shape=jax.ShapeDtypeStruct((rows, LATENT), jnp.float32),
    )(x, w1, b1, w2, b2, w3, b3)


def _edge_step(u, edge3, mk, w1a, w1b, w1c, b1, w2, b2, w3, b3, g, bl):
    half = pl.BlockSpec((1, BE, LATENT), lambda h, i: (h, i, 0))
    swap = pl.BlockSpec((1, BE, LATENT), lambda h, i: (1 - h, i, 0))
    mspec = pl.BlockSpec((BE, 1), lambda h, i: (i, 0))
    return pl.pallas_call(
        _edge_body,
        grid=(2, H_PAD // BE),
        in_specs=[half, swap, half, mspec,
                  _WSPEC2, _WSPEC2, _WSPEC2, _VSPEC2,
                  _WSPEC2, _VSPEC2, _WSPEC2, _VSPEC2, _VSPEC2, _VSPEC2],
        out_specs=[half, half],
        out_shape=[jax.ShapeDtypeStruct((2, H_PAD, LATENT), jnp.float32),
                   jax.ShapeDtypeStruct((2, H_PAD, LATENT), jnp.float32)],
    )(u, u, edge3, mk, w1a, w1b, w1c, b1, w2, b2, w3, b3, g, bl)


def _node_step(node, parts, w1a, w1b, b1, w2, b2, w3, b3, g, bl):
    p0 = pl.BlockSpec((1, BN, LATENT), lambda i: (0, i, 0))
    p1 = pl.BlockSpec((1, BN, LATENT), lambda i: (1, i, 0))
    return pl.pallas_call(
        _node_body,
        grid=(N_PAD // BN,),
        in_specs=[_rowspec(BN), p0, p1,
                  _WSPEC, _WSPEC, _VSPEC,
                  _WSPEC, _VSPEC, _WSPEC, _VSPEC, _VSPEC, _VSPEC],
        out_specs=_rowspec(BN),
        out_shape=jax.ShapeDtypeStruct((N_PAD, LATENT), jnp.float32),
    )(node, parts, parts, w1a, w1b, b1, w2, b2, w3, b3, g, bl)


@functools.cache
def _build_sc_gather():
    mesh = plsc.VectorSubcoreMesh(core_axis_name="c", subcore_axis_name="s")

    @functools.partial(
        pl.kernel,
        mesh=mesh,
        out_type=jax.ShapeDtypeStruct((E_PAD, LATENT), jnp.float32),
        scratch_types=[pltpu.VMEM((GCH, 120), jnp.int32)]
        + [pltpu.VMEM((120, LATENT), jnp.float32)] * NSLOT
        + [pltpu.SemaphoreType.DMA] * (2 * NSLOT),
    )
    def gather_kernel(node_hbm, idx_hbm, out_hbm, idx_v, *rest):
        bufs = rest[:NSLOT]
        gsem = rest[NSLOT:2 * NSLOT]
        wsem = rest[2 * NSLOT:]
        wid = lax.axis_index("s") * 2 + lax.axis_index("c")
        pltpu.sync_copy(idx_hbm.at[wid], idx_v)
        base = wid * GPW

        def _out_at(c):
            return out_hbm.at[pl.ds(base + c * 120, 120)]

        for k in range(NSLOT):
            pltpu.async_copy(node_hbm.at[idx_v.at[k]], bufs[k], gsem[k])

        def cycle(t, carry):
            for k in range(NSLOT):
                c = t * NSLOT + k
                pltpu.make_async_copy(node_hbm.at[idx_v.at[c]], bufs[k],
                                      gsem[k]).wait()
                pltpu.async_copy(bufs[k], _out_at(c), wsem[k])
            for k in range(NSLOT):
                c = t * NSLOT + k
                n = c + NSLOT

                @pl.when(n < GCH)
                def _():
                    pltpu.make_async_copy(bufs[k], _out_at(c), wsem[k]).wait()
                    pltpu.async_copy(node_hbm.at[idx_v.at[n]], bufs[k],
                                     gsem[k])
            return carry

        lax.fori_loop(0, GCH // NSLOT, cycle, 0)
        for k in range(NSLOT):
            pltpu.make_async_copy(bufs[k], _out_at(GCH - NSLOT + k),
                                  wsem[k]).wait()

    return gather_kernel


def _sc_gather(node, gidx):
    return _build_sc_gather()(node, gidx)


@functools.cache
def _build_sc_scatter():
    mesh = plsc.VectorSubcoreMesh(core_axis_name="c", subcore_axis_name="s")

    @functools.partial(
        pl.kernel,
        mesh=mesh,
        out_type=jax.ShapeDtypeStruct((2, N_PAD, LATENT), jnp.float32),
        scratch_types=[pltpu.VMEM((SCH, 120), jnp.int32),
                       pltpu.VMEM_SHARED((N_PAD, LATENT), jnp.float32)]
        + [pltpu.VMEM((120, LATENT), jnp.float32)] * NSLOT_S
        + [pltpu.SemaphoreType.DMA] * (2 * NSLOT_S),
    )
    def scatter_kernel(y_hbm, idx_hbm, out_hbm, idx_v, shared, *rest):
        bufs = rest[:NSLOT_S]
        lsem = rest[NSLOT_S:2 * NSLOT_S]
        ssem = rest[2 * NSLOT_S:]
        cc = lax.axis_index("c")
        s = lax.axis_index("s")
        wid = s * 2 + cc

        def zrow(i, carry):
            for j in range(LATENT // 16):
                bufs[0][i, pl.ds(j * 16, 16)] = jnp.zeros((16,), jnp.float32)
            return carry

        lax.fori_loop(0, 120, zrow, 0)
        z0 = s * ZPT
        zcp = [pltpu.async_copy(bufs[0], shared.at[pl.ds(z0 + off, 120)],
                                lsem[i % NSLOT_S])
               for i, off in enumerate((0, 120, 240, 360, 480, 520))]
        for d in zcp:
            d.wait()
        plsc.subcore_barrier()

        pltpu.sync_copy(idx_hbm.at[wid], idx_v)
        base = wid * SPW

        def _y_at(c):
            return y_hbm.at[pl.ds(base + c * 120, 120)]

        for k in range(NSLOT_S):
            pltpu.async_copy(_y_at(k), bufs[k], lsem[k])

        def cycle(t, carry):
            for k in range(NSLOT_S):
                c = t * NSLOT_S + k

                @pl.when(c < SCH)
                def _():
                    pltpu.make_async_copy(_y_at(c), bufs[k], lsem[k]).wait()
                    pltpu.async_copy(bufs[k], shared.at[idx_v.at[c]],
                                     ssem[k], add=True)
            for k in range(NSLOT_S):
                c = t * NSLOT_S + k
                n = c + NSLOT_S

                @pl.when(n < SCH)
                def _():
                    pltpu.make_async_copy(bufs[k], shared.at[idx_v.at[c]],
                                          ssem[k]).wait()
                    pltpu.async_copy(_y_at(n), bufs[k], lsem[k])
            return carry

        lax.fori_loop(0, (SCH + NSLOT_S - 1) // NSLOT_S, cycle, 0)
        for k in range(NSLOT_S):
            c = SCH - NSLOT_S + k
            pltpu.make_async_copy(bufs[k], shared.at[idx_v.at[c]],
                                  ssem[k]).wait()
        plsc.subcore_barrier()

        ocp = [None] * NSLOT_S
        for i, off in enumerate((0, 120, 240, 360, 480, 520)):
            k = i % NSLOT_S
            if ocp[k] is not None:
                ocp[k].wait()
            pltpu.async_copy(shared.at[pl.ds(z0 + off, 120)],
                             bufs[k], lsem[k]).wait()
            ocp[k] = pltpu.async_copy(
                bufs[k], out_hbm.at[cc, pl.ds(z0 + off, 120)], ssem[k])
        for d in ocp:
            if d is not None:
                d.wait()

    return scatter_kernel


def _sc_scatter(y, sidx):
    return _build_sc_scatter()(y, sidx)


def _tri_edges(cells):
    cells = cells.astype(jnp.int32)
    e = jnp.concatenate([cells[:, 0:2], cells[:, 1:3],
                         jnp.stack([cells[:, 2], cells[:, 0]], axis=1)],
                        axis=0)
    lo = jnp.minimum(e[:, 0], e[:, 1])
    hi = jnp.maximum(e[:, 0], e[:, 1])
    packed = jnp.sort(lo * N_NODES + hi)
    valid = jnp.concatenate([jnp.ones((1,), dtype=bool),
                             packed[1:] != packed[:-1]])
    return packed // N_NODES, packed % N_NODES, valid


def _norm(x, mask=None):
    if mask is None:
        cnt = float(x.shape[0])
        mean = jnp.sum(x, axis=0) / cnt
        var = jnp.sum(x * x, axis=0) / cnt - mean * mean
    else:
        w = mask.astype(x.dtype)[:, None]
        cnt = jnp.sum(mask.astype(x.dtype))
        mean = jnp.sum(x * w, axis=0) / cnt
        var = jnp.sum((x * x) * w, axis=0) / cnt - mean * mean
    std = jnp.maximum(jnp.sqrt(jnp.maximum(var, 0.0)), 1e-8)
    return (x - mean) / std


def _prep_mlp(ps, in_dim=None):
    out = []
    for i, (w, b) in enumerate(ps):
        if i == 0 and in_dim is not None and w.shape[0] != LATENT:
            w = jnp.pad(w, ((0, LATENT - w.shape[0]), (0, 0)))
        if w.shape[1] != LATENT:
            w = jnp.pad(w, ((0, 0), (0, LATENT - w.shape[1])))
            b = jnp.pad(b, (0, LATENT - b.shape[0]))
        out.append(w.astype(jnp.float32))
        out.append(b.reshape(1, -1).astype(jnp.float32))
    return out


def kernel(world_pos, mesh_pos, node_type, cells, params_net, is_training):
    s_half, r_half, valid = _tri_edges(cells)

    one_hot = jax.nn.one_hot(node_type[:, 0], NODE_TYPE_SIZE,
                             dtype=jnp.float32)
    dwp = jnp.take(world_pos, s_half, axis=0) - jnp.take(world_pos, r_half,
                                                         axis=0)
    dmp = jnp.take(mesh_pos, s_half, axis=0) - jnp.take(mesh_pos, r_half,
                                                        axis=0)
    nwp = jnp.linalg.norm(dwp, axis=-1, keepdims=True)
    nmp = jnp.linalg.norm(dmp, axis=-1, keepdims=True)
    ef = jnp.concatenate([
        jnp.concatenate([dwp, nwp, dmp, nmp], axis=-1),
        jnp.concatenate([-dwp, nwp, -dmp, nmp], axis=-1)], axis=0)
    edge_mask = jnp.concatenate([valid, valid])
    ef = _norm(ef, mask=edge_mask)
    nf = _norm(one_hot)

    nf_pad = jnp.zeros((N_PAD, LATENT), jnp.float32)
    nf_pad = nf_pad.at[:N_NODES, :NODE_TYPE_SIZE].set(nf)
    ef_pad = jnp.zeros((2, H_PAD, LATENT), jnp.float32)
    ef_pad = ef_pad.at[0, :H, :8].set(ef[:H])
    ef_pad = ef_pad.at[1, :H, :8].set(ef[H:])

    pad_h = H_PAD - H
    s_pad = jnp.pad(s_half, (0, pad_h)).astype(jnp.int32)
    r_pad = jnp.pad(r_half, (0, pad_h)).astype(jnp.int32)
    gidx = jnp.concatenate([s_pad, r_pad]).reshape(NW, GCH, 120)
    scidx = jnp.concatenate([r_pad, s_pad]).reshape(NW, SCH, 120)
    mask_h = jnp.pad(valid.astype(jnp.float32), (0, pad_h)).reshape(H_PAD, 1)

    n_enc = _prep_mlp(params_net['node_enc']['mlp'], in_dim=NODE_TYPE_SIZE)
    e_enc = _prep_mlp(params_net['edge_enc']['mlp'], in_dim=8)
    ng, nb = params_net['node_enc']['ln']
    eg, eb = params_net['edge_enc']['ln']
    node = _enc(nf_pad, *n_enc, ng.reshape(1, -1), nb.reshape(1, -1), BN)
    edge = _enc(ef_pad.reshape(E_PAD, LATENT), *e_enc, eg.reshape(1, -1),
                eb.reshape(1, -1), BE).reshape(2, H_PAD, LATENT)

    for step in params_net['steps']:
        (ew1, eb1), (ew2, eb2), (ew3, eb3) = step['edge_mlp']
        (nw1, nb1), (nw2, nb2), (nw3, nb3) = step['node_mlp']
        eg_, eb_ = step['edge_ln']
        ng_, nb_ = step['node_ln']

        u = _sc_gather(node, gidx).reshape(2, H_PAD, LATENT)
        edge, y = _edge_step(
            u, edge, mask_h,
            ew1[:LATENT], ew1[LATENT:2 * LATENT], ew1[2 * LATENT:],
            eb1.reshape(1, -1), ew2, eb2.reshape(1, -1), ew3,
            eb3.reshape(1, -1), eg_.reshape(1, -1), eb_.reshape(1, -1))
        parts = _sc_scatter(y.reshape(E_PAD, LATENT), scidx)
        node = _node_step(
            node, parts, nw1[:LATENT], nw1[LATENT:], nb1.reshape(1, -1),
            nw2, nb2.reshape(1, -1), nw3, nb3.reshape(1, -1),
            ng_.reshape(1, -1), nb_.reshape(1, -1))

    d = _prep_mlp(params_net['dec'])
    out = _dec(node, *d, BN)[:N_NODES, :3]
    return out * jnp.asarray(is_training, dtype=out.dtype)


# speedup vs baseline: 1.2940x; 1.0074x over previous
import functools

import jax
import jax.numpy as jnp
from jax import lax
from jax.experimental import pallas as pl
from jax.experimental.pallas import tpu as pltpu
from jax.experimental.pallas import tpu_sc as plsc

N_NODES = 10000
N_CELLS = 10000
NODE_TYPE_SIZE = 9
LATENT = 128
MP_STEPS = 15

N_PAD = 10240
H = 3 * N_CELLS
H_PAD = 30720
E_PAD = 2 * H_PAD
NW = 32
NSLOT = 8

GPW = E_PAD // NW
GCH = GPW // 120
SPW = E_PAD // NW
SCH = SPW // 120
NSLOT_S = 3
ZPT = N_PAD // 16

BN = 2560
BE = 5120


def _dot(a, w):
    return lax.dot_general(a, w, (((1,), (0,)), ((), ())),
                           preferred_element_type=jnp.float32)


def _layernorm(h, g, b):
    m = jnp.mean(h, axis=-1, keepdims=True)
    v = jnp.mean((h - m) ** 2, axis=-1, keepdims=True)
    return (h - m) * lax.rsqrt(v + 1e-5) * g + b


def _enc_body(x_ref, w1, b1, w2, b2, w3, b3, g, bl, o_ref):
    h = jnp.maximum(_dot(x_ref[...], w1[...]) + b1[...], 0.0)
    h = jnp.maximum(_dot(h, w2[...]) + b2[...], 0.0)
    h = _dot(h, w3[...]) + b3[...]
    o_ref[...] = _layernorm(h, g[...], bl[...])


def _dec_body(x_ref, w1, b1, w2, b2, w3, b3, o_ref):
    h = jnp.maximum(_dot(x_ref[...], w1[...]) + b1[...], 0.0)
    h = jnp.maximum(_dot(h, w2[...]) + b2[...], 0.0)
    o_ref[...] = _dot(h, w3[...]) + b3[...]


def _edge_body(ua, ub, e, mk, w1a, w1b, w1c, b1, w2, b2, w3, b3, g, bl,
               ne_ref, y_ref):
    a = ua[0]
    b = ub[0]
    e0 = e[0]
    h = (_dot(a, w1a[...]) + _dot(b, w1b[...]) + _dot(e0, w1c[...]) + b1[...])
    h = jnp.maximum(h, 0.0)
    h = jnp.maximum(_dot(h, w2[...]) + b2[...], 0.0)
    h = _dot(h, w3[...]) + b3[...]
    ne = _layernorm(h, g[...], bl[...]) + e0
    ne_ref[0] = ne
    y_ref[0] = ne * mk[...]


def _node_body(nd, p0, p1, w1a, w1b, b1, w2, b2, w3, b3, g, bl, o_ref):
    ag = p0[0] + p1[0]
    h = jnp.maximum(_dot(nd[...], w1a[...]) + _dot(ag, w1b[...])
                    + b1[...], 0.0)
    h = jnp.maximum(_dot(h, w2[...]) + b2[...], 0.0)
    h = _dot(h, w3[...]) + b3[...]
    o_ref[...] = _layernorm(h, g[...], bl[...]) + nd[...]


def _rowspec(blk):
    return pl.BlockSpec((blk, LATENT), lambda i: (i, 0))


_WSPEC = pl.BlockSpec((LATENT, LATENT), lambda i: (0, 0))
_VSPEC = pl.BlockSpec((1, LATENT), lambda i: (0, 0))
_WSPEC2 = pl.BlockSpec((LATENT, LATENT), lambda h, i: (0, 0))
_VSPEC2 = pl.BlockSpec((1, LATENT), lambda h, i: (0, 0))


def _enc(x, w1, b1, w2, b2, w3, b3, g, bl, blk):
    rows = x.shape[0]
    return pl.pallas_call(
        _enc_body,
        grid=(rows // blk,),
        in_specs=[_rowspec(blk)] + [_WSPEC, _VSPEC] * 3 + [_VSPEC, _VSPEC],
        out_specs=_rowspec(blk),
        out_shape=jax.ShapeDtypeStruct((rows, LATENT), jnp.float32),
    )(x, w1, b1, w2, b2, w3, b3, g, bl)


def _dec(x, w1, b1, w2, b2, w3, b3, blk):
    rows = x.shape[0]
    return pl.pallas_call(
        _dec_body,
        grid=(rows // blk,),
        in_specs=[_rowspec(blk)] + [_WSPEC, _VSPEC] * 3,
        out_specs=_rowspec(blk),
        out_shape=jax.ShapeDtypeStruct((rows, LATENT), jnp.float32),
    )(x, w1, b1, w2, b2, w3, b3)


def _edge_step(u, edge3, mk, w1a, w1b, w1c, b1, w2, b2, w3, b3, g, bl):
    half = pl.BlockSpec((1, BE, LATENT), lambda h, i: (h, i, 0))
    swap = pl.BlockSpec((1, BE, LATENT), lambda h, i: (1 - h, i, 0))
    mspec = pl.BlockSpec((BE, 1), lambda h, i: (i, 0))
    return pl.pallas_call(
        _edge_body,
        grid=(2, H_PAD // BE),
        in_specs=[half, swap, half, mspec,
                  _WSPEC2, _WSPEC2, _WSPEC2, _VSPEC2,
                  _WSPEC2, _VSPEC2, _WSPEC2, _VSPEC2, _VSPEC2, _VSPEC2],
        out_specs=[half, half],
        out_shape=[jax.ShapeDtypeStruct((2, H_PAD, LATENT), jnp.float32),
                   jax.ShapeDtypeStruct((2, H_PAD, LATENT), jnp.float32)],
    )(u, u, edge3, mk, w1a, w1b, w1c, b1, w2, b2, w3, b3, g, bl)


def _node_step(node, parts, w1a, w1b, b1, w2, b2, w3, b3, g, bl):
    p0 = pl.BlockSpec((1, BN, LATENT), lambda i: (0, i, 0))
    p1 = pl.BlockSpec((1, BN, LATENT), lambda i: (1, i, 0))
    return pl.pallas_call(
        _node_body,
        grid=(N_PAD // BN,),
        in_specs=[_rowspec(BN), p0, p1,
                  _WSPEC, _WSPEC, _VSPEC,
                  _WSPEC, _VSPEC, _WSPEC, _VSPEC, _VSPEC, _VSPEC],
        out_specs=_rowspec(BN),
        out_shape=jax.ShapeDtypeStruct((N_PAD, LATENT), jnp.float32),
    )(node, parts, parts, w1a, w1b, b1, w2, b2, w3, b3, g, bl)


@functools.cache
def _build_sc_gather():
    mesh = plsc.VectorSubcoreMesh(core_axis_name="c", subcore_axis_name="s")

    @functools.partial(
        pl.kernel,
        mesh=mesh,
        out_type=jax.ShapeDtypeStruct((E_PAD, LATENT), jnp.float32),
        scratch_types=[pltpu.VMEM((GCH, 120), jnp.int32)]
        + [pltpu.VMEM((120, LATENT), jnp.float32)] * NSLOT
        + [pltpu.SemaphoreType.DMA] * (2 * NSLOT),
    )
    def gather_kernel(node_hbm, idx_hbm, out_hbm, idx_v, *rest):
        bufs = rest[:NSLOT]
        gsem = rest[NSLOT:2 * NSLOT]
        wsem = rest[2 * NSLOT:]
        wid = lax.axis_index("s") * 2 + lax.axis_index("c")
        pltpu.sync_copy(idx_hbm.at[wid], idx_v)
        base = wid * GPW

        def _out_at(c):
            return out_hbm.at[pl.ds(base + c * 120, 120)]

        for k in range(NSLOT):
            pltpu.async_copy(node_hbm.at[idx_v.at[k]], bufs[k], gsem[k])

        def cycle(t, carry):
            for k in range(NSLOT):
                c = t * NSLOT + k
                pltpu.make_async_copy(node_hbm.at[idx_v.at[c]], bufs[k],
                                      gsem[k]).wait()
                pltpu.async_copy(bufs[k], _out_at(c), wsem[k])
            for k in range(NSLOT):
                c = t * NSLOT + k
                n = c + NSLOT

                @pl.when(n < GCH)
                def _():
                    pltpu.make_async_copy(bufs[k], _out_at(c), wsem[k]).wait()
                    pltpu.async_copy(node_hbm.at[idx_v.at[n]], bufs[k],
                                     gsem[k])
            return carry

        lax.fori_loop(0, GCH // NSLOT, cycle, 0)
        for k in range(NSLOT):
            pltpu.make_async_copy(bufs[k], _out_at(GCH - NSLOT + k),
                                  wsem[k]).wait()

    return gather_kernel


def _sc_gather(node, gidx):
    return _build_sc_gather()(node, gidx)


@functools.cache
def _build_sc_scatter():
    mesh = plsc.VectorSubcoreMesh(core_axis_name="c", subcore_axis_name="s")

    @functools.partial(
        pl.kernel,
        mesh=mesh,
        out_type=jax.ShapeDtypeStruct((2, N_PAD, LATENT), jnp.float32),
        scratch_types=[pltpu.VMEM((SCH, 120), jnp.int32),
                       pltpu.VMEM_SHARED((N_PAD, LATENT), jnp.float32)]
        + [pltpu.VMEM((120, LATENT), jnp.float32)] * NSLOT_S
        + [pltpu.SemaphoreType.DMA] * (2 * NSLOT_S),
    )
    def scatter_kernel(y_hbm, idx_hbm, out_hbm, idx_v, shared, *rest):
        bufs = rest[:NSLOT_S]
        lsem = rest[NSLOT_S:2 * NSLOT_S]
        ssem = rest[2 * NSLOT_S:]
        cc = lax.axis_index("c")
        s = lax.axis_index("s")
        wid = s * 2 + cc

        def zrow(i, carry):
            for j in range(LATENT // 16):
                bufs[0][i, pl.ds(j * 16, 16)] = jnp.zeros((16,), jnp.float32)
            return carry

        lax.fori_loop(0, 120, zrow, 0)
        z0 = s * ZPT
        zcp = [pltpu.async_copy(bufs[0], shared.at[pl.ds(z0 + off, 120)],
                                lsem[i % NSLOT_S])
               for i, off in enumerate((0, 120, 240, 360, 480, 520))]
        for d in zcp:
            d.wait()
        plsc.subcore_barrier()

        pltpu.sync_copy(idx_hbm.at[wid], idx_v)
        base = wid * SPW

        def _y_at(c):
            return y_hbm.at[pl.ds(base + c * 120, 120)]

        for k in range(NSLOT_S):
            pltpu.async_copy(_y_at(k), bufs[k], lsem[k])

        def cycle(t, carry):
            for k in range(NSLOT_S):
                c = t * NSLOT_S + k

                @pl.when(c < SCH)
                def _():
                    pltpu.make_async_copy(_y_at(c), bufs[k], lsem[k]).wait()
                    pltpu.async_copy(bufs[k], shared.at[idx_v.at[c]],
                                     ssem[k], add=True)
            for k in range(NSLOT_S):
                c = t * NSLOT_S + k
                n = c + NSLOT_S

                @pl.when(n < SCH)
                def _():
                    pltpu.make_async_copy(bufs[k], shared.at[idx_v.at[c]],
                                          ssem[k]).wait()
                    pltpu.async_copy(_y_at(n), bufs[k], lsem[k])
            return carry

        lax.fori_loop(0, (SCH + NSLOT_S - 1) // NSLOT_S, cycle, 0)
        for k in range(NSLOT_S):
            c = SCH - NSLOT_S + k
            pltpu.make_async_copy(bufs[k], shared.at[idx_v.at[c]],
                                  ssem[k]).wait()
        plsc.subcore_barrier()

        ocp = [None] * NSLOT_S
        for i, off in enumerate((0, 120, 240, 360, 480, 520)):
            k = i % NSLOT_S
            if ocp[k] is not None:
                ocp[k].wait()
            pltpu.async_copy(shared.at[pl.ds(z0 + off, 120)],
                             bufs[k], lsem[k]).wait()
            ocp[k] = pltpu.async_copy(
                bufs[k], out_hbm.at[cc, pl.ds(z0 + off, 120)], ssem[k])
        for d in ocp:
            if d is not None:
                d.wait()

    return scatter_kernel


def _sc_scatter(y, sidx):
    return _build_sc_scatter()(y, sidx)


def _tri_edges(cells):
    cells = cells.astype(jnp.int32)
    e = jnp.concatenate([cells[:, 0:2], cells[:, 1:3],
                         jnp.stack([cells[:, 2], cells[:, 0]], axis=1)],
                        axis=0)
    lo = jnp.minimum(e[:, 0], e[:, 1])
    hi = jnp.maximum(e[:, 0], e[:, 1])
    packed = jnp.sort(lo * N_NODES + hi)
    valid = jnp.concatenate([jnp.ones((1,), dtype=bool),
                             packed[1:] != packed[:-1]])
    return packed // N_NODES, packed % N_NODES, valid


def _norm(x, mask=None):
    if mask is None:
        cnt = float(x.shape[0])
        mean = jnp.sum(x, axis=0) / cnt
        var = jnp.sum(x * x, axis=0) / cnt - mean * mean
    else:
        w = mask.astype(x.dtype)[:, None]
        cnt = jnp.sum(mask.astype(x.dtype))
        mean = jnp.sum(x * w, axis=0) / cnt
        var = jnp.sum((x * x) * w, axis=0) / cnt - mean * mean
    std = jnp.maximum(jnp.sqrt(jnp.maximum(var, 0.0)), 1e-8)
    return (x - mean) / std


def _prep_mlp(ps, in_dim=None):
    out = []
    for i, (w, b) in enumerate(ps):
        if i == 0 and in_dim is not None and w.shape[0] != LATENT:
            w = jnp.pad(w, ((0, LATENT - w.shape[0]), (0, 0)))
        if w.shape[1] != LATENT:
            w = jnp.pad(w, ((0, 0), (0, LATENT - w.shape[1])))
            b = jnp.pad(b, (0, LATENT - b.shape[0]))
        out.append(w.astype(jnp.float32))
        out.append(b.reshape(1, -1).astype(jnp.float32))
    return out


def kernel(world_pos, mesh_pos, node_type, cells, params_net, is_training):
    s_half, r_half, valid = _tri_edges(cells)

    one_hot = jax.nn.one_hot(node_type[:, 0], NODE_TYPE_SIZE,
                             dtype=jnp.float32)
    dwp = jnp.take(world_pos, s_half, axis=0) - jnp.take(world_pos, r_half,
                                                         axis=0)
    dmp = jnp.take(mesh_pos, s_half, axis=0) - jnp.take(mesh_pos, r_half,
                                                        axis=0)
    nwp = jnp.linalg.norm(dwp, axis=-1, keepdims=True)
    nmp = jnp.linalg.norm(dmp, axis=-1, keepdims=True)
    ef = jnp.concatenate([
        jnp.concatenate([dwp, nwp, dmp, nmp], axis=-1),
        jnp.concatenate([-dwp, nwp, -dmp, nmp], axis=-1)], axis=0)
    edge_mask = jnp.concatenate([valid, valid])
    ef = _norm(ef, mask=edge_mask)
    nf = _norm(one_hot)

    nf_pad = jnp.zeros((N_PAD, LATENT), jnp.float32)
    nf_pad = nf_pad.at[:N_NODES, :NODE_TYPE_SIZE].set(nf)
    ef_pad = jnp.zeros((2, H_PAD, LATENT), jnp.float32)
    ef_pad = ef_pad.at[0, :H, :8].set(ef[:H])
    ef_pad = ef_pad.at[1, :H, :8].set(ef[H:])

    pad_h = H_PAD - H
    s_pad = jnp.pad(s_half, (0, pad_h)).astype(jnp.int32)
    r_pad = jnp.pad(r_half, (0, pad_h)).astype(jnp.int32)
    gidx = jnp.concatenate([s_pad, r_pad]).reshape(NW, GCH, 120)
    scidx = jnp.concatenate([r_pad, s_pad]).reshape(NW, SCH, 120)
    mask_h = jnp.pad(valid.astype(jnp.float32), (0, pad_h)).reshape(H_PAD, 1)

    n_enc = _prep_mlp(params_net['node_enc']['mlp'], in_dim=NODE_TYPE_SIZE)
    e_enc = _prep_mlp(params_net['edge_enc']['mlp'], in_dim=8)
    ng, nb = params_net['node_enc']['ln']
    eg, eb = params_net['edge_enc']['ln']
    node = _enc(nf_pad, *n_enc, ng.reshape(1, -1), nb.reshape(1, -1), BN)
    edge = _enc(ef_pad.reshape(E_PAD, LATENT), *e_enc, eg.reshape(1, -1),
                eb.reshape(1, -1), BE).reshape(2, H_PAD, LATENT)

    for step in params_net['steps']:
        (ew1, eb1), (ew2, eb2), (ew3, eb3) = step['edge_mlp']
        (nw1, nb1), (nw2, nb2), (nw3, nb3) = step['node_mlp']
        eg_, eb_ = step['edge_ln']
        ng_, nb_ = step['node_ln']

        u = _sc_gather(node, gidx).reshape(2, H_PAD, LATENT)
        edge, y = _edge_step(
            u, edge, mask_h,
            ew1[:LATENT], ew1[LATENT:2 * LATENT], ew1[2 * LATENT:],
            eb1.reshape(1, -1), ew2, eb2.reshape(1, -1), ew3,
            eb3.reshape(1, -1), eg_.reshape(1, -1), eb_.reshape(1, -1))
        parts = _sc_scatter(y.reshape(E_PAD, LATENT), scidx)
        node = _node_step(
            node, parts, nw1[:LATENT], nw1[LATENT:], nb1.reshape(1, -1),
            nw2, nb2.reshape(1, -1), nw3, nb3.reshape(1, -1),
            ng_.reshape(1, -1), nb_.reshape(1, -1))

    d = _prep_mlp(params_net['dec'])
    out = _dec(node, *d, BN)[:N_NODES, :3]
    return out * jnp.asarray(is_training, dtype=out.dtype)


# speedup vs baseline: 1.3107x; 1.0129x over previous
import functools

import jax
import jax.numpy as jnp
from jax import lax
from jax.experimental import pallas as pl
from jax.experimental.pallas import tpu as pltpu
from jax.experimental.pallas import tpu_sc as plsc

N_NODES = 10000
N_CELLS = 10000
NODE_TYPE_SIZE = 9
LATENT = 128
MP_STEPS = 15

N_PAD = 10240
H = 3 * N_CELLS
H_PAD = 30720
E_PAD = 2 * H_PAD
NW = 32
NSLOT = 8

GPW = E_PAD // NW
GCH = GPW // 120
SPW = E_PAD // NW
SCH = SPW // 120
NSLOT_S = 3
ZPT = N_PAD // 16

BN = 2560
BE = 5120


def _dot(a, w):
    return lax.dot_general(a, w, (((1,), (0,)), ((), ())),
                           preferred_element_type=jnp.float32)


def _layernorm(h, g, b):
    m = jnp.mean(h, axis=-1, keepdims=True)
    v = jnp.mean((h - m) ** 2, axis=-1, keepdims=True)
    return (h - m) * lax.rsqrt(v + 1e-5) * g + b


def _enc_body(x_ref, w1, b1, w2, b2, w3, b3, g, bl, o_ref):
    h = jnp.maximum(_dot(x_ref[...], w1[...]) + b1[...], 0.0)
    h = jnp.maximum(_dot(h, w2[...]) + b2[...], 0.0)
    h = _dot(h, w3[...]) + b3[...]
    o_ref[...] = _layernorm(h, g[...], bl[...])


def _dec_body(x_ref, w1, b1, w2, b2, w3, b3, o_ref):
    h = jnp.maximum(_dot(x_ref[...], w1[...]) + b1[...], 0.0)
    h = jnp.maximum(_dot(h, w2[...]) + b2[...], 0.0)
    o_ref[...] = _dot(h, w3[...]) + b3[...]


def _edge_body(ua, ub, e, mk, w1a, w1b, w1c, b1, w2, b2, w3, b3, g, bl,
               ne_ref, y_ref):
    a = ua[0]
    b = ub[0]
    e0 = e[0]
    h = (_dot(a, w1a[...]) + _dot(b, w1b[...]) + _dot(e0, w1c[...]) + b1[...])
    h = jnp.maximum(h, 0.0)
    h = jnp.maximum(_dot(h, w2[...]) + b2[...], 0.0)
    h = _dot(h, w3[...]) + b3[...]
    ne = _layernorm(h, g[...], bl[...]) + e0
    ne_ref[0] = ne
    y_ref[0] = ne * mk[...]


def _node_body(nd, p0, p1, w1a, w1b, b1, w2, b2, w3, b3, g, bl, o_ref):
    ag = p0[0] + p1[0]
    h = jnp.maximum(_dot(nd[...], w1a[...]) + _dot(ag, w1b[...])
                    + b1[...], 0.0)
    h = jnp.maximum(_dot(h, w2[...]) + b2[...], 0.0)
    h = _dot(h, w3[...]) + b3[...]
    o_ref[...] = _layernorm(h, g[...], bl[...]) + nd[...]


def _rowspec(blk):
    return pl.BlockSpec((blk, LATENT), lambda i: (i, 0))


_WSPEC = pl.BlockSpec((LATENT, LATENT), lambda i: (0, 0))
_VSPEC = pl.BlockSpec((1, LATENT), lambda i: (0, 0))
_WSPEC2 = pl.BlockSpec((LATENT, LATENT), lambda h, i: (0, 0))
_VSPEC2 = pl.BlockSpec((1, LATENT), lambda h, i: (0, 0))


def _enc(x, w1, b1, w2, b2, w3, b3, g, bl, blk):
    rows = x.shape[0]
    return pl.pallas_call(
        _enc_body,
        grid=(rows // blk,),
        in_specs=[_rowspec(blk)] + [_WSPEC, _VSPEC] * 3 + [_VSPEC, _VSPEC],
        out_specs=_rowspec(blk),
        out_shape=jax.ShapeDtypeStruct((rows, LATENT), jnp.float32),
    )(x, w1, b1, w2, b2, w3, b3, g, bl)


def _dec(x, w1, b1, w2, b2, w3, b3, blk):
    rows = x.shape[0]
    return pl.pallas_call(
        _dec_body,
        grid=(rows // blk,),
        in_specs=[_rowspec(blk)] + [_WSPEC, _VSPEC] * 3,
        out_specs=_rowspec(blk),
        out_shape=jax.ShapeDtypeStruct((rows, LATENT), jnp.float32),
    )(x, w1, b1, w2, b2, w3, b3)


def _edge_step(u, edge3, mk, w1a, w1b, w1c, b1, w2, b2, w3, b3, g, bl):
    half = pl.BlockSpec((1, BE, LATENT), lambda h, i: (h, i, 0))
    swap = pl.BlockSpec((1, BE, LATENT), lambda h, i: (1 - h, i, 0))
    mspec = pl.BlockSpec((BE, 1), lambda h, i: (i, 0))
    return pl.pallas_call(
        _edge_body,
        grid=(2, H_PAD // BE),
        in_specs=[half, swap, half, mspec,
                  _WSPEC2, _WSPEC2, _WSPEC2, _VSPEC2,
                  _WSPEC2, _VSPEC2, _WSPEC2, _VSPEC2, _VSPEC2, _VSPEC2],
        out_specs=[half, half],
        out_shape=[jax.ShapeDtypeStruct((2, H_PAD, LATENT), jnp.float32),
                   jax.ShapeDtypeStruct((2, H_PAD, LATENT), jnp.float32)],
    )(u, u, edge3, mk, w1a, w1b, w1c, b1, w2, b2, w3, b3, g, bl)


def _node_step(node, parts, w1a, w1b, b1, w2, b2, w3, b3, g, bl):
    p0 = pl.BlockSpec((1, BN, LATENT), lambda i: (0, i, 0))
    p1 = pl.BlockSpec((1, BN, LATENT), lambda i: (1, i, 0))
    return pl.pallas_call(
        _node_body,
        grid=(N_PAD // BN,),
        in_specs=[_rowspec(BN), p0, p1,
                  _WSPEC, _WSPEC, _VSPEC,
                  _WSPEC, _VSPEC, _WSPEC, _VSPEC, _VSPEC, _VSPEC],
        out_specs=_rowspec(BN),
        out_shape=jax.ShapeDtypeStruct((N_PAD, LATENT), jnp.float32),
    )(node, parts, parts, w1a, w1b, b1, w2, b2, w3, b3, g, bl)


@functools.cache
def _build_sc_gather():
    mesh = plsc.VectorSubcoreMesh(core_axis_name="c", subcore_axis_name="s")

    @functools.partial(
        pl.kernel,
        mesh=mesh,
        out_type=jax.ShapeDtypeStruct((E_PAD, LATENT), jnp.float32),
        scratch_types=[pltpu.VMEM((GCH, 120), jnp.int32)]
        + [pltpu.VMEM((120, LATENT), jnp.float32)] * NSLOT
        + [pltpu.SemaphoreType.DMA] * (2 * NSLOT),
    )
    def gather_kernel(node_hbm, idx_hbm, out_hbm, idx_v, *rest):
        bufs = rest[:NSLOT]
        gsem = rest[NSLOT:2 * NSLOT]
        wsem = rest[2 * NSLOT:]
        wid = lax.axis_index("s") * 2 + lax.axis_index("c")
        pltpu.sync_copy(idx_hbm.at[wid], idx_v)
        base = wid * GPW

        def _out_at(c):
            return out_hbm.at[pl.ds(base + c * 120, 120)]

        for k in range(NSLOT):
            pltpu.async_copy(node_hbm.at[idx_v.at[k]], bufs[k], gsem[k])

        def cycle(t, carry):
            for k in range(NSLOT):
                c = t * NSLOT + k
                pltpu.make_async_copy(node_hbm.at[idx_v.at[c]], bufs[k],
                                      gsem[k]).wait()
                pltpu.async_copy(bufs[k], _out_at(c), wsem[k])
            for k in range(NSLOT):
                c = t * NSLOT + k
                n = c + NSLOT

                @pl.when(n < GCH)
                def _():
                    pltpu.make_async_copy(bufs[k], _out_at(c), wsem[k]).wait()
                    pltpu.async_copy(node_hbm.at[idx_v.at[n]], bufs[k],
                                     gsem[k])
            return carry

        lax.fori_loop(0, GCH // NSLOT, cycle, 0)
        for k in range(NSLOT):
            pltpu.make_async_copy(bufs[k], _out_at(GCH - NSLOT + k),
                                  wsem[k]).wait()

    return gather_kernel


def _sc_gather(node, gidx):
    return _build_sc_gather()(node, gidx)


@functools.cache
def _build_sc_scatter():
    mesh = plsc.VectorSubcoreMesh(core_axis_name="c", subcore_axis_name="s")

    @functools.partial(
        pl.kernel,
        mesh=mesh,
        out_type=jax.ShapeDtypeStruct((2, N_PAD, LATENT), jnp.float32),
        scratch_types=[pltpu.VMEM((SCH, 120), jnp.int32),
                       pltpu.VMEM_SHARED((N_PAD, LATENT), jnp.float32)]
        + [pltpu.VMEM((120, LATENT), jnp.float32)] * NSLOT_S
        + [pltpu.SemaphoreType.DMA] * (2 * NSLOT_S),
    )
    def scatter_kernel(y_hbm, idx_hbm, out_hbm, idx_v, shared, *rest):
        bufs = rest[:NSLOT_S]
        lsem = rest[NSLOT_S:2 * NSLOT_S]
        ssem = rest[2 * NSLOT_S:]
        cc = lax.axis_index("c")
        s = lax.axis_index("s")
        wid = s * 2 + cc

        base = wid * SPW

        def _y_at(c):
            return y_hbm.at[pl.ds(base + c * 120, 120)]

        pltpu.async_copy(idx_hbm.at[wid], idx_v, ssem[2])
        pltpu.async_copy(_y_at(0), bufs[0], lsem[0])
        pltpu.async_copy(_y_at(1), bufs[1], lsem[1])

        def zrow(i, carry):
            for j in range(LATENT // 16):
                bufs[2][i, pl.ds(j * 16, 16)] = jnp.zeros((16,), jnp.float32)
            return carry

        lax.fori_loop(0, 120, zrow, 0)
        z0 = s * ZPT
        zcp = [pltpu.async_copy(bufs[2], shared.at[pl.ds(z0 + off, 120)],
                                ssem[i % 2])
               for i, off in enumerate((0, 120, 240, 360, 480, 520))]
        for d in zcp:
            d.wait()
        plsc.subcore_barrier()

        pltpu.make_async_copy(idx_hbm.at[wid], idx_v, ssem[2]).wait()
        pltpu.async_copy(_y_at(2), bufs[2], lsem[2])

        def cycle(t, carry):
            for k in range(NSLOT_S):
                c = t * NSLOT_S + k

                @pl.when(c < SCH)
                def _():
                    pltpu.make_async_copy(_y_at(c), bufs[k], lsem[k]).wait()
                    pltpu.async_copy(bufs[k], shared.at[idx_v.at[c]],
                                     ssem[k], add=True)
            for k in range(NSLOT_S):
                c = t * NSLOT_S + k
                n = c + NSLOT_S

                @pl.when(n < SCH)
                def _():
                    pltpu.make_async_copy(bufs[k], shared.at[idx_v.at[c]],
                                          ssem[k]).wait()
                    pltpu.async_copy(_y_at(n), bufs[k], lsem[k])
            return carry

        lax.fori_loop(0, (SCH + NSLOT_S - 1) // NSLOT_S, cycle, 0)
        for k in range(NSLOT_S):
            c = SCH - NSLOT_S + k
            pltpu.make_async_copy(bufs[k], shared.at[idx_v.at[c]],
                                  ssem[k]).wait()
        plsc.subcore_barrier()

        ocp = [None] * NSLOT_S
        for i, off in enumerate((0, 120, 240, 360, 480, 520)):
            k = i % NSLOT_S
            if ocp[k] is not None:
                ocp[k].wait()
            pltpu.async_copy(shared.at[pl.ds(z0 + off, 120)],
                             bufs[k], lsem[k]).wait()
            ocp[k] = pltpu.async_copy(
                bufs[k], out_hbm.at[cc, pl.ds(z0 + off, 120)], ssem[k])
        for d in ocp:
            if d is not None:
                d.wait()

    return scatter_kernel


def _sc_scatter(y, sidx):
    return _build_sc_scatter()(y, sidx)


def _tri_edges(cells):
    cells = cells.astype(jnp.int32)
    e = jnp.concatenate([cells[:, 0:2], cells[:, 1:3],
                         jnp.stack([cells[:, 2], cells[:, 0]], axis=1)],
                        axis=0)
    lo = jnp.minimum(e[:, 0], e[:, 1])
    hi = jnp.maximum(e[:, 0], e[:, 1])
    packed = jnp.sort(lo * N_NODES + hi)
    valid = jnp.concatenate([jnp.ones((1,), dtype=bool),
                             packed[1:] != packed[:-1]])
    return packed // N_NODES, packed % N_NODES, valid


def _norm(x, mask=None):
    if mask is None:
        cnt = float(x.shape[0])
        mean = jnp.sum(x, axis=0) / cnt
        var = jnp.sum(x * x, axis=0) / cnt - mean * mean
    else:
        w = mask.astype(x.dtype)[:, None]
        cnt = jnp.sum(mask.astype(x.dtype))
        mean = jnp.sum(x * w, axis=0) / cnt
        var = jnp.sum((x * x) * w, axis=0) / cnt - mean * mean
    std = jnp.maximum(jnp.sqrt(jnp.maximum(var, 0.0)), 1e-8)
    return (x - mean) / std


def _prep_mlp(ps, in_dim=None):
    out = []
    for i, (w, b) in enumerate(ps):
        if i == 0 and in_dim is not None and w.shape[0] != LATENT:
            w = jnp.pad(w, ((0, LATENT - w.shape[0]), (0, 0)))
        if w.shape[1] != LATENT:
            w = jnp.pad(w, ((0, 0), (0, LATENT - w.shape[1])))
            b = jnp.pad(b, (0, LATENT - b.shape[0]))
        out.append(w.astype(jnp.float32))
        out.append(b.reshape(1, -1).astype(jnp.float32))
    return out


def kernel(world_pos, mesh_pos, node_type, cells, params_net, is_training):
    s_half, r_half, valid = _tri_edges(cells)

    one_hot = jax.nn.one_hot(node_type[:, 0], NODE_TYPE_SIZE,
                             dtype=jnp.float32)
    dwp = jnp.take(world_pos, s_half, axis=0) - jnp.take(world_pos, r_half,
                                                         axis=0)
    dmp = jnp.take(mesh_pos, s_half, axis=0) - jnp.take(mesh_pos, r_half,
                                                        axis=0)
    nwp = jnp.linalg.norm(dwp, axis=-1, keepdims=True)
    nmp = jnp.linalg.norm(dmp, axis=-1, keepdims=True)
    ef = jnp.concatenate([
        jnp.concatenate([dwp, nwp, dmp, nmp], axis=-1),
        jnp.concatenate([-dwp, nwp, -dmp, nmp], axis=-1)], axis=0)
    edge_mask = jnp.concatenate([valid, valid])
    ef = _norm(ef, mask=edge_mask)
    nf = _norm(one_hot)

    nf_pad = jnp.zeros((N_PAD, LATENT), jnp.float32)
    nf_pad = nf_pad.at[:N_NODES, :NODE_TYPE_SIZE].set(nf)
    ef_pad = jnp.zeros((2, H_PAD, LATENT), jnp.float32)
    ef_pad = ef_pad.at[0, :H, :8].set(ef[:H])
    ef_pad = ef_pad.at[1, :H, :8].set(ef[H:])

    pad_h = H_PAD - H
    s_pad = jnp.pad(s_half, (0, pad_h)).astype(jnp.int32)
    r_pad = jnp.pad(r_half, (0, pad_h)).astype(jnp.int32)
    gidx = jnp.concatenate([s_pad, r_pad]).reshape(NW, GCH, 120)
    scidx = jnp.concatenate([r_pad, s_pad]).reshape(NW, SCH, 120)
    mask_h = jnp.pad(valid.astype(jnp.float32), (0, pad_h)).reshape(H_PAD, 1)

    n_enc = _prep_mlp(params_net['node_enc']['mlp'], in_dim=NODE_TYPE_SIZE)
    e_enc = _prep_mlp(params_net['edge_enc']['mlp'], in_dim=8)
    ng, nb = params_net['node_enc']['ln']
    eg, eb = params_net['edge_enc']['ln']
    node = _enc(nf_pad, *n_enc, ng.reshape(1, -1), nb.reshape(1, -1), BN)
    edge = _enc(ef_pad.reshape(E_PAD, LATENT), *e_enc, eg.reshape(1, -1),
                eb.reshape(1, -1), BE).reshape(2, H_PAD, LATENT)

    for step in params_net['steps']:
        (ew1, eb1), (ew2, eb2), (ew3, eb3) = step['edge_mlp']
        (nw1, nb1), (nw2, nb2), (nw3, nb3) = step['node_mlp']
        eg_, eb_ = step['edge_ln']
        ng_, nb_ = step['node_ln']

        u = _sc_gather(node, gidx).reshape(2, H_PAD, LATENT)
        edge, y = _edge_step(
            u, edge, mask_h,
            ew1[:LATENT], ew1[LATENT:2 * LATENT], ew1[2 * LATENT:],
            eb1.reshape(1, -1), ew2, eb2.reshape(1, -1), ew3,
            eb3.reshape(1, -1), eg_.reshape(1, -1), eb_.reshape(1, -1))
        parts = _sc_scatter(y.reshape(E_PAD, LATENT), scidx)
        node = _node_step(
            node, parts, nw1[:LATENT], nw1[LATENT:], nb1.reshape(1, -1),
            nw2, nb2.reshape(1, -1), nw3, nb3.reshape(1, -1),
            ng_.reshape(1, -1), nb_.reshape(1, -1))

    d = _prep_mlp(params_net['dec'])
    out = _dec(node, *d, BN)[:N_NODES, :3]
    return out * jnp.asarray(is_training, dtype=out.dtype)
